# Initial kernel scaffold; baseline (speedup 1.0000x reference)
import functools

import jax
import jax.numpy as jnp
from jax import lax
from jax.experimental import pallas as pl
from jax.experimental.pallas import tpu as pltpu
from jax.experimental.pallas import tpu_sc as plsc

N = 10000
E = 320000
F_IN = 128
H = 64
R = 16
NC = 6

NCORES = 2
NSUB = 16
NW = NCORES * NSUB

LANES = 16
CW = 80
EPW = E // NW
ROWS_PW = EPW // CW
WIN_ROWS = 5
WIN_E = WIN_ROWS * CW
NWIN = ROWS_PW // WIN_ROWS

EPS_A = E // NSUB
ROWS_A = EPS_A // CW
WINA_ROWS = 25
NWIN_A = ROWS_A // WINA_ROWS

NR = N * R
CNT_STRIPE = NR // NSUB
ACC_STRIPE = N // NSUB

_mesh = plsc.VectorSubcoreMesh(core_axis_name="c", subcore_axis_name="s",
                               num_cores=NCORES, num_subcores=NSUB)


def _f32(*shape):
    return jax.ShapeDtypeStruct(shape, jnp.float32)


def _tca_body(x_ref, w2_ref, root_ref, b1_ref, xr_ref, ro_ref):
    xb = x_ref[...]
    xr_ref[...] = jnp.dot(xb, w2_ref[...], preferred_element_type=jnp.float32)
    ro_ref[...] = (jnp.dot(xb, root_ref[...], preferred_element_type=jnp.float32)
                   + b1_ref[...])


def _tc_a(x, w2, root, b1):
    bn = 1000
    grid = (N // bn,)
    return pl.pallas_call(
        _tca_body,
        grid=grid,
        in_specs=[
            pl.BlockSpec((bn, F_IN), lambda i: (i, 0)),
            pl.BlockSpec((F_IN, R * H), lambda i: (0, 0)),
            pl.BlockSpec((F_IN, H), lambda i: (0, 0)),
            pl.BlockSpec((1, H), lambda i: (0, 0)),
        ],
        out_specs=[
            pl.BlockSpec((bn, R * H), lambda i: (i, 0)),
            pl.BlockSpec((bn, H), lambda i: (i, 0)),
        ],
        out_shape=[_f32(N, R * H), _f32(N, H)],
    )(x, w2, root, b1)


def _sc1_body(xr_hbm, src_hbm, dst_hbm, et_hbm, zc_hbm, za_hbm, ones_hbm,
              p0_hbm, p1_hbm,
              cnt_sp, acc_sp,
              srcw, dstw, etw, gidxw, keyw, cntv, invf, rows, onesv, sem):
    c = lax.axis_index("c")
    s = lax.axis_index("s")
    wid = c * NSUB + s

    pltpu.sync_copy(zc_hbm, cnt_sp.at[pl.ds(s * CNT_STRIPE, CNT_STRIPE)])
    pltpu.sync_copy(za_hbm, acc_sp.at[pl.ds(s * ACC_STRIPE, ACC_STRIPE)])
    pltpu.sync_copy(ones_hbm, onesv)
    plsc.subcore_barrier()

    a_base = s * ROWS_A

    @pl.loop(0, NWIN_A)
    def _hist_win(w):
        rb = a_base + w * WINA_ROWS
        pltpu.sync_copy(dst_hbm.at[pl.ds(rb, WINA_ROWS)], dstw)
        pltpu.sync_copy(et_hbm.at[pl.ds(rb, WINA_ROWS)], etw)
        for j in range(WINA_ROWS):
            for q in range(CW // LANES):
                sl = pl.ds(q * LANES, LANES)
                keyw[j, sl] = dstw[j, sl] * R + etw[j, sl]
        for j in range(WINA_ROWS):
            pltpu.sync_copy(onesv, cnt_sp.at[keyw.at[j]], add=True)

    plsc.subcore_barrier()

    b_base = wid * ROWS_PW

    @pl.loop(0, NWIN)
    def _edge_win(w):
        rb = b_base + w * WIN_ROWS
        pltpu.sync_copy(src_hbm.at[pl.ds(rb, WIN_ROWS)], srcw)
        pltpu.sync_copy(dst_hbm.at[pl.ds(rb, WIN_ROWS)],
                        dstw.at[pl.ds(0, WIN_ROWS)])
        pltpu.sync_copy(et_hbm.at[pl.ds(rb, WIN_ROWS)],
                        etw.at[pl.ds(0, WIN_ROWS)])
        for j in range(WIN_ROWS):
            for q in range(CW // LANES):
                sl = pl.ds(q * LANES, LANES)
                gidxw[j, sl] = srcw[j, sl] * R + etw[j, sl]
                keyw[j, sl] = dstw[j, sl] * R + etw[j, sl]
        for j in range(WIN_ROWS):
            pltpu.sync_copy(cnt_sp.at[keyw.at[j]], cntv.at[j])
        for j in range(WIN_ROWS):
            for q in range(CW // LANES):
                sl = pl.ds(q * LANES, LANES)
                invf[pl.ds(j * CW + q * LANES, LANES)] = (
                    1.0 / jnp.maximum(cntv[j, sl], 1.0))
        cps = []
        for j in range(WIN_ROWS):
            cps.append(pltpu.async_copy(
                xr_hbm.at[gidxw.at[j]], rows.at[pl.ds(j * CW, CW)], sem))
        for cp in cps:
            cp.wait()

        @pl.loop(0, WIN_E)
        def _scale(e):
            nv = plsc.load_gather(invf, [jnp.broadcast_to(e, (LANES,))])
            for q in range(H // LANES):
                sl = pl.ds(q * LANES, LANES)
                rows[e, sl] = rows[e, sl] * nv

        for j in range(WIN_ROWS):
            pltpu.sync_copy(rows.at[pl.ds(j * CW, CW)],
                            acc_sp.at[dstw.at[j]], add=True)

    plsc.subcore_barrier()

    stripe = pl.ds(s * ACC_STRIPE, ACC_STRIPE)

    @pl.when(c == 0)
    def _():
        pltpu.sync_copy(acc_sp.at[stripe], p0_hbm.at[stripe])

    @pl.when(c == 1)
    def _():
        pltpu.sync_copy(acc_sp.at[stripe], p1_hbm.at[stripe])


def _sc_1(xr2, src2d, dst2d, et2d, zc, za, ones):
    kern = pl.kernel(
        _sc1_body,
        out_type=[_f32(N, H), _f32(N, H)],
        mesh=_mesh,
        scratch_types=[
            pltpu.VMEM_SHARED((NR,), jnp.float32),
            pltpu.VMEM_SHARED((N, H), jnp.float32),
            pltpu.VMEM((WIN_ROWS, CW), jnp.int32),
            pltpu.VMEM((WINA_ROWS, CW), jnp.int32),
            pltpu.VMEM((WINA_ROWS, CW), jnp.int32),
            pltpu.VMEM((WIN_ROWS, CW), jnp.int32),
            pltpu.VMEM((WINA_ROWS, CW), jnp.int32),
            pltpu.VMEM((WIN_ROWS, CW), jnp.float32),
            pltpu.VMEM((WIN_E,), jnp.float32),
            pltpu.VMEM((WIN_E, H), jnp.float32),
            pltpu.VMEM((CW,), jnp.float32),
            pltpu.SemaphoreType.DMA,
        ],
    )
    return kern(xr2, src2d, dst2d, et2d, zc, za, ones)


def _sc2_body(p0_hbm, p1_hbm, ro_hbm, src_hbm, dst_hbm, za_hbm,
              out1_hbm, q0_hbm, q1_hbm,
              tab_sp, acc_sp,
              srcw, dstw, rows, ta, tb, tc_, sem):
    c = lax.axis_index("c")
    s = lax.axis_index("s")
    wid = c * NSUB + s

    pltpu.sync_copy(za_hbm, acc_sp.at[pl.ds(s * ACC_STRIPE, ACC_STRIPE)])
    chunk = 125
    for k in range(ACC_STRIPE // chunk):
        rb = s * ACC_STRIPE + k * chunk
        pltpu.sync_copy(p0_hbm.at[pl.ds(rb, chunk)], ta)
        pltpu.sync_copy(p1_hbm.at[pl.ds(rb, chunk)], tb)
        pltpu.sync_copy(ro_hbm.at[pl.ds(rb, chunk)], tc_)
        for i in range(chunk):
            for q in range(H // LANES):
                sl = pl.ds(q * LANES, LANES)
                ta[i, sl] = ta[i, sl] + tb[i, sl] + tc_[i, sl]
        pltpu.sync_copy(ta, tab_sp.at[pl.ds(rb, chunk)])

        @pl.when(c == 0)
        def _():
            pltpu.sync_copy(ta, out1_hbm.at[pl.ds(rb, chunk)])

    plsc.subcore_barrier()

    b_base = wid * ROWS_PW

    @pl.loop(0, NWIN)
    def _edge_win(w):
        rb = b_base + w * WIN_ROWS
        pltpu.sync_copy(src_hbm.at[pl.ds(rb, WIN_ROWS)], srcw)
        pltpu.sync_copy(dst_hbm.at[pl.ds(rb, WIN_ROWS)], dstw)
        for j in range(WIN_ROWS):
            pltpu.sync_copy(tab_sp.at[srcw.at[j]], rows.at[pl.ds(j * CW, CW)])
        for j in range(WIN_ROWS):
            pltpu.sync_copy(rows.at[pl.ds(j * CW, CW)],
                            acc_sp.at[dstw.at[j]], add=True)

    plsc.subcore_barrier()

    stripe = pl.ds(s * ACC_STRIPE, ACC_STRIPE)

    @pl.when(c == 0)
    def _():
        pltpu.sync_copy(acc_sp.at[stripe], q0_hbm.at[stripe])

    @pl.when(c == 1)
    def _():
        pltpu.sync_copy(acc_sp.at[stripe], q1_hbm.at[stripe])


def _sc_2(p0, p1, ro, src2d, dst2d, za):
    kern = pl.kernel(
        _sc2_body,
        out_type=[_f32(N, H), _f32(N, H), _f32(N, H)],
        mesh=_mesh,
        scratch_types=[
            pltpu.VMEM_SHARED((N, H), jnp.float32),
            pltpu.VMEM_SHARED((N, H), jnp.float32),
            pltpu.VMEM((WIN_ROWS, CW), jnp.int32),
            pltpu.VMEM((WIN_ROWS, CW), jnp.int32),
            pltpu.VMEM((WIN_E, H), jnp.float32),
            pltpu.VMEM((125, H), jnp.float32),
            pltpu.VMEM((125, H), jnp.float32),
            pltpu.VMEM((125, H), jnp.float32),
            pltpu.SemaphoreType.DMA,
        ],
    )
    return kern(p0, p1, ro, src2d, dst2d, za)


def _tcb_body(x_ref, o1_ref, q0_ref, q1_ref, wrel_ref, brel_ref, wr2_ref,
              wlx_ref, wlo_ref, bl_ref, ws_ref, bs_ref, out_ref):
    agg = q0_ref[...] + q1_ref[...]
    out1 = o1_ref[...]
    out2 = (jnp.dot(agg, wrel_ref[...], preferred_element_type=jnp.float32)
            + jnp.dot(out1, wr2_ref[...], preferred_element_type=jnp.float32)
            + brel_ref[...])
    h = jnp.maximum(
        jnp.dot(x_ref[...], wlx_ref[...], preferred_element_type=jnp.float32)
        + jnp.dot(out2, wlo_ref[...], preferred_element_type=jnp.float32)
        + bl_ref[...], 0.0)
    lg = jnp.dot(h, ws_ref[...], preferred_element_type=jnp.float32) + bs_ref[...]
    m = jnp.max(lg, axis=1, keepdims=True)
    sh = lg - m
    out_ref[...] = sh - jnp.log(jnp.sum(jnp.exp(sh), axis=1, keepdims=True))


def _tc_b(x, out1, q0, q1, wrel, brel, wr2, wlx, wlo, bl, ws, bs):
    bn = 1000
    grid = (N // bn,)

    def full(shape):
        return pl.BlockSpec(shape, lambda i: tuple(0 for _ in shape))

    return pl.pallas_call(
        _tcb_body,
        grid=grid,
        in_specs=[
            pl.BlockSpec((bn, F_IN), lambda i: (i, 0)),
            pl.BlockSpec((bn, H), lambda i: (i, 0)),
            pl.BlockSpec((bn, H), lambda i: (i, 0)),
            pl.BlockSpec((bn, H), lambda i: (i, 0)),
            full((H, H)),
            full((1, H)),
            full((H, H)),
            full((F_IN, H)),
            full((H, H)),
            full((1, H)),
            full((H, NC)),
            full((1, NC)),
        ],
        out_specs=pl.BlockSpec((bn, NC), lambda i: (i, 0)),
        out_shape=_f32(N, NC),
    )(x, out1, q0, q1, wrel, brel, wr2, wlx, wlo, bl, ws, bs)


def kernel(x, edge_index, edge_norm, edge_type, seq_lengths, umask,
           nodal_attn, avec, bases, comp, root, bias1, W_rel, b_rel,
           W_root2, Wl, bl, Ws, bs):
    w2 = jnp.einsum("rb,bio->iro", comp, bases).reshape(F_IN, R * H)
    src2d = edge_index[0].astype(jnp.int32).reshape(E // CW, CW)
    dst2d = edge_index[1].astype(jnp.int32).reshape(E // CW, CW)
    et2d = edge_type.astype(jnp.int32).reshape(E // CW, CW)
    zc = jnp.zeros((CNT_STRIPE,), jnp.float32)
    za = jnp.zeros((ACC_STRIPE, H), jnp.float32)
    ones = jnp.ones((CW,), jnp.float32)

    xr, root_out = _tc_a(x, w2, root, bias1.reshape(1, H))
    xr2 = xr.reshape(N * R, H)

    p0, p1 = _sc_1(xr2, src2d, dst2d, et2d, zc, za, ones)

    out1, q0, q1 = _sc_2(p0, p1, root_out, src2d, dst2d, za)

    return _tc_b(x, out1, q0, q1, W_rel, b_rel.reshape(1, H), W_root2,
                 Wl[:F_IN], Wl[F_IN:], bl.reshape(1, H), Ws,
                 bs.reshape(1, NC))


# speedup vs baseline: 22.3821x; 22.3821x over previous
import dataclasses

import jax
import jax.numpy as jnp
from jax import lax
from jax.experimental import pallas as pl
from jax.experimental.pallas import tpu as pltpu
from jax.experimental.pallas import tpu_sc as plsc

N = 10000
E = 320000
F_IN = 128
H = 64
R = 16
NC = 6

NCORES = 2
NSUB = 16
NW = NCORES * NSUB

LANES = 16
CW = 80
WIN_ROWS = 5
WIN_E = WIN_ROWS * CW
NWINDOWS = E // WIN_E
WPW = NWINDOWS // NW
WPS = NWINDOWS // NSUB

NR = N * R
CNT_STRIPE = NR // NSUB
ACC_STRIPE = N // NSUB
GR = 125
NG = N // GR
GPS = NG // NSUB

_mesh = plsc.VectorSubcoreMesh(core_axis_name="c", subcore_axis_name="s",
                               num_cores=NCORES, num_subcores=NSUB)

_sc_params = pltpu.CompilerParams(needs_layout_passes=False,
                                  use_tc_tiling_on_sc=False)


def _f32(*shape):
    return jax.ShapeDtypeStruct(shape, jnp.float32)


def _tca_body(x_ref, w2_ref, root_ref, b1_ref, xr_ref, ro_ref):
    xb = x_ref[...]
    xr_ref[...] = jnp.dot(xb, w2_ref[...], preferred_element_type=jnp.float32)
    ro_ref[...] = (jnp.dot(xb, root_ref[...], preferred_element_type=jnp.float32)
                   + b1_ref[...])


def _tc_a(x, w2, root, b1):
    bn = 1000
    grid = (N // bn,)
    return pl.pallas_call(
        _tca_body,
        grid=grid,
        in_specs=[
            pl.BlockSpec((bn, F_IN), lambda i: (i, 0)),
            pl.BlockSpec((F_IN, R * H), lambda i: (0, 0)),
            pl.BlockSpec((F_IN, H), lambda i: (0, 0)),
            pl.BlockSpec((1, H), lambda i: (0, 0)),
        ],
        out_specs=[
            pl.BlockSpec((bn, R * H), lambda i: (i, 0)),
            pl.BlockSpec((bn, H), lambda i: (i, 0)),
        ],
        out_shape=[_f32(N, R * H), _f32(N, H)],
    )(x, w2, root, b1)


def _sc1_body(xr_hbm, src_hbm, dst_hbm, et_hbm,
              p0_hbm, p1_hbm,
              cnt_sp, acc_sp,
              srcw, dstw, etw, gidxw, keyw, cntv, invf, rows, onesv, sem):
    c = lax.axis_index("c")
    s = lax.axis_index("s")
    wid = c * NSUB + s

    z16 = jnp.zeros((LANES,), jnp.float32)
    o16 = jnp.ones((LANES,), jnp.float32)

    @pl.loop(0, GR)
    def _zrow(i):
        for q in range(H // LANES):
            rows[i, pl.ds(q * LANES, LANES)] = z16

    @pl.loop(0, WIN_E // LANES)
    def _zinv(m):
        invf[pl.ds(m * LANES, LANES)] = z16

    for q in range(CW // LANES):
        onesv[pl.ds(q * LANES, LANES)] = o16

    for k in range(GPS):
        pltpu.sync_copy(rows.at[pl.ds(0, GR)],
                        acc_sp.at[pl.ds(s * ACC_STRIPE + k * GR, GR)])

    @pl.loop(0, CNT_STRIPE // WIN_E)
    def _zcnt(m):
        pltpu.sync_copy(invf,
                        cnt_sp.at[pl.ds(s * CNT_STRIPE + m * WIN_E, WIN_E)])

    plsc.subcore_barrier()


    @pl.loop(s * WPS, (s + 1) * WPS)
    def _hist_win(t):
        pltpu.sync_copy(dst_hbm.at[t], dstw)
        pltpu.sync_copy(et_hbm.at[t], etw)
        for j in range(WIN_ROWS):
            for q in range(CW // LANES):
                sl = pl.ds(q * LANES, LANES)
                keyw[j, sl] = dstw[j, sl] * R + etw[j, sl]
        for j in range(WIN_ROWS):
            pltpu.sync_copy(onesv, cnt_sp.at[keyw.at[j]], add=True)

    plsc.subcore_barrier()


    @pl.loop(wid * WPW, (wid + 1) * WPW)
    def _edge_win(t):
        pltpu.sync_copy(src_hbm.at[t], srcw)
        pltpu.sync_copy(dst_hbm.at[t], dstw)
        pltpu.sync_copy(et_hbm.at[t], etw)
        for j in range(WIN_ROWS):
            for q in range(CW // LANES):
                sl = pl.ds(q * LANES, LANES)
                gidxw[j, sl] = srcw[j, sl] * R + etw[j, sl]
                keyw[j, sl] = dstw[j, sl] * R + etw[j, sl]
        for j in range(WIN_ROWS):
            pltpu.sync_copy(cnt_sp.at[keyw.at[j]], cntv.at[j])
        for j in range(WIN_ROWS):
            for q in range(CW // LANES):
                sl = pl.ds(q * LANES, LANES)
                invf[pl.ds(j * CW + q * LANES, LANES)] = (
                    1.0 / jnp.maximum(cntv[j, sl], 1.0))
        cps = []
        for j in range(WIN_ROWS):
            cps.append(pltpu.async_copy(
                xr_hbm.at[gidxw.at[j]], rows.at[pl.ds(j * CW, CW)], sem))
        for cp in cps:
            cp.wait()

        @pl.loop(0, WIN_E)
        def _scale(e):
            nv = plsc.load_gather(invf, [jnp.broadcast_to(e, (LANES,))])
            for q in range(H // LANES):
                sl = pl.ds(q * LANES, LANES)
                rows[e, sl] = rows[e, sl] * nv

        for j in range(WIN_ROWS):
            pltpu.sync_copy(rows.at[pl.ds(j * CW, CW)],
                            acc_sp.at[dstw.at[j]], add=True)

    plsc.subcore_barrier()

    for k in range(GPS):
        src_sl = pl.ds(s * ACC_STRIPE + k * GR, GR)
        stg = rows.at[pl.ds(0, GR)]
        pltpu.sync_copy(acc_sp.at[src_sl], stg)

        @pl.when(c == 0)
        def _():
            pltpu.sync_copy(stg, p0_hbm.at[s * GPS + k])

        @pl.when(c == 1)
        def _():
            pltpu.sync_copy(stg, p1_hbm.at[s * GPS + k])


def _sc_1(xr2, src3, dst3, et3):
    kern = pl.kernel(
        _sc1_body,
        out_type=[_f32(NG, GR, H), _f32(NG, GR, H)],
        mesh=_mesh,
        scratch_types=[
            pltpu.VMEM_SHARED((NR,), jnp.float32),
            pltpu.VMEM_SHARED((N, H), jnp.float32),
            pltpu.VMEM((WIN_ROWS, CW), jnp.int32),
            pltpu.VMEM((WIN_ROWS, CW), jnp.int32),
            pltpu.VMEM((WIN_ROWS, CW), jnp.int32),
            pltpu.VMEM((WIN_ROWS, CW), jnp.int32),
            pltpu.VMEM((WIN_ROWS, CW), jnp.int32),
            pltpu.VMEM((WIN_ROWS, CW), jnp.float32),
            pltpu.VMEM((WIN_E,), jnp.float32),
            pltpu.VMEM((WIN_E, H), jnp.float32),
            pltpu.VMEM((CW,), jnp.float32),
            pltpu.SemaphoreType.DMA,
        ],
        compiler_params=_sc_params,
    )
    return kern(xr2, src3, dst3, et3)


def _sc2_body(p0_hbm, p1_hbm, ro_hbm, src_hbm, dst_hbm,
              out1_hbm, q0_hbm, q1_hbm,
              tab_sp, acc_sp,
              srcw, dstw, rows, ta, tb, tc_, sem):
    c = lax.axis_index("c")
    s = lax.axis_index("s")
    wid = c * NSUB + s

    z16 = jnp.zeros((LANES,), jnp.float32)

    @pl.loop(0, GR)
    def _zrow(i):
        for q in range(H // LANES):
            rows[i, pl.ds(q * LANES, LANES)] = z16

    for k in range(GPS):
        pltpu.sync_copy(rows.at[pl.ds(0, GR)],
                        acc_sp.at[pl.ds(s * ACC_STRIPE + k * GR, GR)])

    for k in range(GPS):
        g = s * GPS + k
        pltpu.sync_copy(p0_hbm.at[g], ta)
        pltpu.sync_copy(p1_hbm.at[g], tb)
        pltpu.sync_copy(ro_hbm.at[g], tc_)

        @pl.loop(0, GR)
        def _addrow(i):
            for q in range(H // LANES):
                sl = pl.ds(q * LANES, LANES)
                ta[i, sl] = ta[i, sl] + tb[i, sl] + tc_[i, sl]
        pltpu.sync_copy(ta, tab_sp.at[pl.ds(g * GR, GR)])

        @pl.when(c == 0)
        def _():
            pltpu.sync_copy(ta, out1_hbm.at[g])

    plsc.subcore_barrier()


    @pl.loop(wid * WPW, (wid + 1) * WPW)
    def _edge_win(t):
        pltpu.sync_copy(src_hbm.at[t], srcw)
        pltpu.sync_copy(dst_hbm.at[t], dstw)
        for j in range(WIN_ROWS):
            pltpu.sync_copy(tab_sp.at[srcw.at[j]], rows.at[pl.ds(j * CW, CW)])
        for j in range(WIN_ROWS):
            pltpu.sync_copy(rows.at[pl.ds(j * CW, CW)],
                            acc_sp.at[dstw.at[j]], add=True)

    plsc.subcore_barrier()

    for k in range(GPS):
        src_sl = pl.ds(s * ACC_STRIPE + k * GR, GR)
        pltpu.sync_copy(acc_sp.at[src_sl], ta)

        @pl.when(c == 0)
        def _():
            pltpu.sync_copy(ta, q0_hbm.at[s * GPS + k])

        @pl.when(c == 1)
        def _():
            pltpu.sync_copy(ta, q1_hbm.at[s * GPS + k])


def _sc_2(p0, p1, ro, src3, dst3):
    kern = pl.kernel(
        _sc2_body,
        out_type=[_f32(NG, GR, H), _f32(NG, GR, H), _f32(NG, GR, H)],
        mesh=_mesh,
        scratch_types=[
            pltpu.VMEM_SHARED((N, H), jnp.float32),
            pltpu.VMEM_SHARED((N, H), jnp.float32),
            pltpu.VMEM((WIN_ROWS, CW), jnp.int32),
            pltpu.VMEM((WIN_ROWS, CW), jnp.int32),
            pltpu.VMEM((WIN_E, H), jnp.float32),
            pltpu.VMEM((GR, H), jnp.float32),
            pltpu.VMEM((GR, H), jnp.float32),
            pltpu.VMEM((GR, H), jnp.float32),
            pltpu.SemaphoreType.DMA,
        ],
        compiler_params=_sc_params,
    )
    return kern(p0, p1, ro, src3, dst3)


def _tcb_body(x_ref, o1_ref, q0_ref, q1_ref, wrel_ref, brel_ref, wr2_ref,
              wlx_ref, wlo_ref, bl_ref, ws_ref, bs_ref, out_ref):
    agg = q0_ref[...] + q1_ref[...]
    out1 = o1_ref[...]
    out2 = (jnp.dot(agg, wrel_ref[...], preferred_element_type=jnp.float32)
            + jnp.dot(out1, wr2_ref[...], preferred_element_type=jnp.float32)
            + brel_ref[...])
    h = jnp.maximum(
        jnp.dot(x_ref[...], wlx_ref[...], preferred_element_type=jnp.float32)
        + jnp.dot(out2, wlo_ref[...], preferred_element_type=jnp.float32)
        + bl_ref[...], 0.0)
    lg = jnp.dot(h, ws_ref[...], preferred_element_type=jnp.float32) + bs_ref[...]
    m = jnp.max(lg, axis=1, keepdims=True)
    sh = lg - m
    out_ref[...] = sh - jnp.log(jnp.sum(jnp.exp(sh), axis=1, keepdims=True))


def _tc_b(x, out1, q0, q1, wrel, brel, wr2, wlx, wlo, bl, ws, bs):
    bn = 1000
    grid = (N // bn,)

    def full(shape):
        return pl.BlockSpec(shape, lambda i: tuple(0 for _ in shape))

    return pl.pallas_call(
        _tcb_body,
        grid=grid,
        in_specs=[
            pl.BlockSpec((bn, F_IN), lambda i: (i, 0)),
            pl.BlockSpec((bn, H), lambda i: (i, 0)),
            pl.BlockSpec((bn, H), lambda i: (i, 0)),
            pl.BlockSpec((bn, H), lambda i: (i, 0)),
            full((H, H)),
            full((1, H)),
            full((H, H)),
            full((F_IN, H)),
            full((H, H)),
            full((1, H)),
            full((H, NC)),
            full((1, NC)),
        ],
        out_specs=pl.BlockSpec((bn, NC), lambda i: (i, 0)),
        out_shape=_f32(N, NC),
    )(x, out1, q0, q1, wrel, brel, wr2, wlx, wlo, bl, ws, bs)


def kernel(x, edge_index, edge_norm, edge_type, seq_lengths, umask,
           nodal_attn, avec, bases, comp, root, bias1, W_rel, b_rel,
           W_root2, Wl, bl, Ws, bs):
    w2 = jnp.einsum("rb,bio->iro", comp, bases).reshape(F_IN, R * H)
    src3 = edge_index[0].astype(jnp.int32).reshape(NWINDOWS, WIN_ROWS, CW)
    dst3 = edge_index[1].astype(jnp.int32).reshape(NWINDOWS, WIN_ROWS, CW)
    et3 = edge_type.astype(jnp.int32).reshape(NWINDOWS, WIN_ROWS, CW)

    xr, root_out = _tc_a(x, w2, root, bias1.reshape(1, H))
    xr2 = xr.reshape(N * R, H)
    rog = root_out.reshape(NG, GR, H)

    p0, p1 = _sc_1(xr2, src3, dst3, et3)

    out1g, q0, q1 = _sc_2(p0, p1, rog, src3, dst3)
    out1 = out1g.reshape(N, H)

    return _tc_b(x, out1, q0.reshape(N, H), q1.reshape(N, H), W_rel,
                 b_rel.reshape(1, H), W_root2, Wl[:F_IN], Wl[F_IN:],
                 bl.reshape(1, H), Ws, bs.reshape(1, NC))


# speedup vs baseline: 32.5085x; 1.4524x over previous
import dataclasses

import jax
import jax.numpy as jnp
from jax import lax
from jax.experimental import pallas as pl
from jax.experimental.pallas import tpu as pltpu
from jax.experimental.pallas import tpu_sc as plsc

N = 10000
E = 320000
F_IN = 128
H = 64
R = 16
NC = 6

NCORES = 2
NSUB = 16
NW = NCORES * NSUB

LANES = 16
CW = 80
WIN_ROWS = 5
WIN_E = WIN_ROWS * CW
NWINDOWS = E // WIN_E
WPW = NWINDOWS // NW
WPS = NWINDOWS // NSUB

NR = N * R
CNT_STRIPE = NR // NSUB
ACC_STRIPE = N // NSUB
GR = 125
NG = N // GR
GPS = NG // NSUB

_mesh = plsc.VectorSubcoreMesh(core_axis_name="c", subcore_axis_name="s",
                               num_cores=NCORES, num_subcores=NSUB)

_sc_params = pltpu.CompilerParams(needs_layout_passes=False,
                                  use_tc_tiling_on_sc=False)


def _f32(*shape):
    return jax.ShapeDtypeStruct(shape, jnp.float32)


def _tca_body(x_ref, w2_ref, root_ref, b1_ref, xr_ref, ro_ref):
    xb = x_ref[...]
    xr_ref[...] = jnp.dot(xb, w2_ref[...], preferred_element_type=jnp.float32)
    ro_ref[...] = (jnp.dot(xb, root_ref[...], preferred_element_type=jnp.float32)
                   + b1_ref[...])


def _tc_a(x, w2, root, b1):
    bn = 1000
    grid = (N // bn,)
    return pl.pallas_call(
        _tca_body,
        grid=grid,
        in_specs=[
            pl.BlockSpec((bn, F_IN), lambda i: (i, 0)),
            pl.BlockSpec((F_IN, R * H), lambda i: (0, 0)),
            pl.BlockSpec((F_IN, H), lambda i: (0, 0)),
            pl.BlockSpec((1, H), lambda i: (0, 0)),
        ],
        out_specs=[
            pl.BlockSpec((bn, R * H), lambda i: (i, 0)),
            pl.BlockSpec((bn, H), lambda i: (i, 0)),
        ],
        out_shape=[_f32(N, R * H), _f32(N, H)],
    )(x, w2, root, b1)


def _sc1_body(xr_hbm, sde_hbm,
              p0_hbm, p1_hbm,
              cnt_sp, acc_sp,
              sdew, sdea, keya0, keya1, gidxw, keyw, cntv, invf, rows,
              onesv, semi, semg, semc, sems, sema0, sema1):
    c = lax.axis_index("c")
    s = lax.axis_index("s")
    wid = c * NSUB + s

    z16 = jnp.zeros((LANES,), jnp.float32)
    o16 = jnp.ones((LANES,), jnp.float32)

    @pl.loop(0, GR)
    def _zrow(i):
        for q in range(H // LANES):
            rows[i, pl.ds(q * LANES, LANES)] = z16

    @pl.loop(0, WIN_E // LANES)
    def _zinv(m):
        invf[pl.ds(m * LANES, LANES)] = z16

    for q in range(CW // LANES):
        onesv[pl.ds(q * LANES, LANES)] = o16

    for k in range(GPS):
        pltpu.sync_copy(rows.at[pl.ds(0, GR)],
                        acc_sp.at[pl.ds(s * ACC_STRIPE + k * GR, GR)])

    @pl.loop(0, CNT_STRIPE // WIN_E)
    def _zcnt(m):
        pltpu.sync_copy(invf,
                        cnt_sp.at[pl.ds(s * CNT_STRIPE + m * WIN_E, WIN_E)])

    plsc.subcore_barrier()


    @pl.loop(0, WPS // 2)
    def _hist_pair(i):
        t = s * WPS + 2 * i
        pltpu.sync_copy(sde_hbm.at[pl.ds(t, 2)], sdea)
        for b, keyb, semb in ((0, keya0, sema0), (1, keya1, sema1)):
            @pl.when(i > 0)
            def _():
                for j in range(WIN_ROWS):
                    pltpu.make_async_copy(
                        onesv, cnt_sp.at[keyb.at[j]], semb).wait()
            for j in range(WIN_ROWS):
                for q in range(CW // LANES):
                    sl = pl.ds(q * LANES, LANES)
                    keyb[j, sl] = sdea[b, 1, j, sl] * R + sdea[b, 2, j, sl]
            for j in range(WIN_ROWS):
                pltpu.async_copy(onesv, cnt_sp.at[keyb.at[j]], semb,
                                 add=True)

    for keyb, semb in ((keya0, sema0), (keya1, sema1)):
        for j in range(WIN_ROWS):
            pltpu.make_async_copy(onesv, cnt_sp.at[keyb.at[j]], semb).wait()

    plsc.subcore_barrier()


    @pl.loop(wid * WPW, (wid + 1) * WPW)
    def _edge_win(t):
        pltpu.sync_copy(sde_hbm.at[t], sdew)
        for j in range(WIN_ROWS):
            for q in range(CW // LANES):
                sl = pl.ds(q * LANES, LANES)
                gidxw[j, sl] = sdew[0, j, sl] * R + sdew[2, j, sl]
                keyw[j, sl] = sdew[1, j, sl] * R + sdew[2, j, sl]
        gcps = [pltpu.async_copy(
            xr_hbm.at[gidxw.at[j]], rows.at[pl.ds(j * CW, CW)], semg)
            for j in range(WIN_ROWS)]
        ccps = [pltpu.async_copy(cnt_sp.at[keyw.at[j]], cntv.at[j], semc)
                for j in range(WIN_ROWS)]
        for cp in ccps:
            cp.wait()
        for j in range(WIN_ROWS):
            for q in range(CW // LANES):
                sl = pl.ds(q * LANES, LANES)
                invf[pl.ds(j * CW + q * LANES, LANES)] = (
                    1.0 / jnp.maximum(cntv[j, sl], 1.0))
        for cp in gcps:
            cp.wait()

        @plsc.parallel_loop(0, WIN_E, unroll=4)
        def _scale(e):
            nv = plsc.load_gather(invf, [jnp.broadcast_to(e, (LANES,))])
            for q in range(H // LANES):
                sl = pl.ds(q * LANES, LANES)
                rows[e, sl] = rows[e, sl] * nv

        scps = [pltpu.async_copy(
            rows.at[pl.ds(j * CW, CW)], acc_sp.at[sdew.at[1].at[j]], sems,
            add=True) for j in range(WIN_ROWS)]
        for cp in scps:
            cp.wait()

    plsc.subcore_barrier()

    for k in range(GPS):
        src_sl = pl.ds(s * ACC_STRIPE + k * GR, GR)
        stg = rows.at[pl.ds(0, GR)]
        pltpu.sync_copy(acc_sp.at[src_sl], stg)

        @pl.when(c == 0)
        def _():
            pltpu.sync_copy(stg, p0_hbm.at[s * GPS + k])

        @pl.when(c == 1)
        def _():
            pltpu.sync_copy(stg, p1_hbm.at[s * GPS + k])


def _sc_1(xr2, sde):
    kern = pl.kernel(
        _sc1_body,
        out_type=[_f32(NG, GR, H), _f32(NG, GR, H)],
        mesh=_mesh,
        scratch_types=[
            pltpu.VMEM_SHARED((NR,), jnp.float32),
            pltpu.VMEM_SHARED((N, H), jnp.float32),
            pltpu.VMEM((3, WIN_ROWS, CW), jnp.int32),
            pltpu.VMEM((2, 3, WIN_ROWS, CW), jnp.int32),
            pltpu.VMEM((WIN_ROWS, CW), jnp.int32),
            pltpu.VMEM((WIN_ROWS, CW), jnp.int32),
            pltpu.VMEM((WIN_ROWS, CW), jnp.int32),
            pltpu.VMEM((WIN_ROWS, CW), jnp.int32),
            pltpu.VMEM((WIN_ROWS, CW), jnp.float32),
            pltpu.VMEM((WIN_E,), jnp.float32),
            pltpu.VMEM((WIN_E, H), jnp.float32),
            pltpu.VMEM((CW,), jnp.float32),
            pltpu.SemaphoreType.DMA,
            pltpu.SemaphoreType.DMA,
            pltpu.SemaphoreType.DMA,
            pltpu.SemaphoreType.DMA,
            pltpu.SemaphoreType.DMA,
            pltpu.SemaphoreType.DMA,
        ],
        compiler_params=_sc_params,
    )
    return kern(xr2, sde)


def _sc2_body(p0_hbm, p1_hbm, ro_hbm, sde_hbm,
              out1_hbm, q0_hbm, q1_hbm,
              tab_sp, acc_sp,
              sdew, rows, ta, tb, tc_, semg, sems):
    c = lax.axis_index("c")
    s = lax.axis_index("s")
    wid = c * NSUB + s

    z16 = jnp.zeros((LANES,), jnp.float32)

    @pl.loop(0, GR)
    def _zrow(i):
        for q in range(H // LANES):
            rows[i, pl.ds(q * LANES, LANES)] = z16

    for k in range(GPS):
        pltpu.sync_copy(rows.at[pl.ds(0, GR)],
                        acc_sp.at[pl.ds(s * ACC_STRIPE + k * GR, GR)])

    for k in range(GPS):
        g = s * GPS + k
        pltpu.sync_copy(p0_hbm.at[g], ta)
        pltpu.sync_copy(p1_hbm.at[g], tb)
        pltpu.sync_copy(ro_hbm.at[g], tc_)

        @pl.loop(0, GR)
        def _addrow(i):
            for q in range(H // LANES):
                sl = pl.ds(q * LANES, LANES)
                ta[i, sl] = ta[i, sl] + tb[i, sl] + tc_[i, sl]
        pltpu.sync_copy(ta, tab_sp.at[pl.ds(g * GR, GR)])

        @pl.when(c == 0)
        def _():
            pltpu.sync_copy(ta, out1_hbm.at[g])

    plsc.subcore_barrier()


    @pl.loop(wid * WPW, (wid + 1) * WPW)
    def _edge_win(t):
        pltpu.sync_copy(sde_hbm.at[t], sdew)
        gcps = [pltpu.async_copy(
            tab_sp.at[sdew.at[0].at[j]], rows.at[pl.ds(j * CW, CW)], semg)
            for j in range(WIN_ROWS)]
        for cp in gcps:
            cp.wait()
        scps = [pltpu.async_copy(
            rows.at[pl.ds(j * CW, CW)], acc_sp.at[sdew.at[1].at[j]], sems,
            add=True) for j in range(WIN_ROWS)]
        for cp in scps:
            cp.wait()

    plsc.subcore_barrier()

    for k in range(GPS):
        src_sl = pl.ds(s * ACC_STRIPE + k * GR, GR)
        pltpu.sync_copy(acc_sp.at[src_sl], ta)

        @pl.when(c == 0)
        def _():
            pltpu.sync_copy(ta, q0_hbm.at[s * GPS + k])

        @pl.when(c == 1)
        def _():
            pltpu.sync_copy(ta, q1_hbm.at[s * GPS + k])


def _sc_2(p0, p1, ro, sde):
    kern = pl.kernel(
        _sc2_body,
        out_type=[_f32(NG, GR, H), _f32(NG, GR, H), _f32(NG, GR, H)],
        mesh=_mesh,
        scratch_types=[
            pltpu.VMEM_SHARED((N, H), jnp.float32),
            pltpu.VMEM_SHARED((N, H), jnp.float32),
            pltpu.VMEM((3, WIN_ROWS, CW), jnp.int32),
            pltpu.VMEM((WIN_E, H), jnp.float32),
            pltpu.VMEM((GR, H), jnp.float32),
            pltpu.VMEM((GR, H), jnp.float32),
            pltpu.VMEM((GR, H), jnp.float32),
            pltpu.SemaphoreType.DMA,
            pltpu.SemaphoreType.DMA,
        ],
        compiler_params=_sc_params,
    )
    return kern(p0, p1, ro, sde)


def _tcb_body(x_ref, o1_ref, q0_ref, q1_ref, wrel_ref, brel_ref, wr2_ref,
              wlx_ref, wlo_ref, bl_ref, ws_ref, bs_ref, out_ref):
    agg = q0_ref[...] + q1_ref[...]
    out1 = o1_ref[...]
    out2 = (jnp.dot(agg, wrel_ref[...], preferred_element_type=jnp.float32)
            + jnp.dot(out1, wr2_ref[...], preferred_element_type=jnp.float32)
            + brel_ref[...])
    h = jnp.maximum(
        jnp.dot(x_ref[...], wlx_ref[...], preferred_element_type=jnp.float32)
        + jnp.dot(out2, wlo_ref[...], preferred_element_type=jnp.float32)
        + bl_ref[...], 0.0)
    lg = jnp.dot(h, ws_ref[...], preferred_element_type=jnp.float32) + bs_ref[...]
    m = jnp.max(lg, axis=1, keepdims=True)
    sh = lg - m
    out_ref[...] = sh - jnp.log(jnp.sum(jnp.exp(sh), axis=1, keepdims=True))


def _tc_b(x, out1, q0, q1, wrel, brel, wr2, wlx, wlo, bl, ws, bs):
    bn = 1000
    grid = (N // bn,)

    def full(shape):
        return pl.BlockSpec(shape, lambda i: tuple(0 for _ in shape))

    return pl.pallas_call(
        _tcb_body,
        grid=grid,
        in_specs=[
            pl.BlockSpec((bn, F_IN), lambda i: (i, 0)),
            pl.BlockSpec((bn, H), lambda i: (i, 0)),
            pl.BlockSpec((bn, H), lambda i: (i, 0)),
            pl.BlockSpec((bn, H), lambda i: (i, 0)),
            full((H, H)),
            full((1, H)),
            full((H, H)),
            full((F_IN, H)),
            full((H, H)),
            full((1, H)),
            full((H, NC)),
            full((1, NC)),
        ],
        out_specs=pl.BlockSpec((bn, NC), lambda i: (i, 0)),
        out_shape=_f32(N, NC),
    )(x, out1, q0, q1, wrel, brel, wr2, wlx, wlo, bl, ws, bs)


def kernel(x, edge_index, edge_norm, edge_type, seq_lengths, umask,
           nodal_attn, avec, bases, comp, root, bias1, W_rel, b_rel,
           W_root2, Wl, bl, Ws, bs):
    w2 = jnp.einsum("rb,bio->iro", comp, bases).reshape(F_IN, R * H)
    src3 = edge_index[0].astype(jnp.int32).reshape(NWINDOWS, WIN_ROWS, CW)
    dst3 = edge_index[1].astype(jnp.int32).reshape(NWINDOWS, WIN_ROWS, CW)
    et3 = edge_type.astype(jnp.int32).reshape(NWINDOWS, WIN_ROWS, CW)
    sde = jnp.stack([src3, dst3, et3], axis=1)

    xr, root_out = _tc_a(x, w2, root, bias1.reshape(1, H))
    xr2 = xr.reshape(N * R, H)
    rog = root_out.reshape(NG, GR, H)

    p0, p1 = _sc_1(xr2, sde)

    out1g, q0, q1 = _sc_2(p0, p1, rog, sde)
    out1 = out1g.reshape(N, H)

    return _tc_b(x, out1, q0.reshape(N, H), q1.reshape(N, H), W_rel,
                 b_rel.reshape(1, H), W_root2, Wl[:F_IN], Wl[F_IN:],
                 bl.reshape(1, H), Ws, bs.reshape(1, NC))


# speedup vs baseline: 41.9398x; 1.2901x over previous
import jax
import jax.numpy as jnp
from jax import lax
from jax.experimental import pallas as pl
from jax.experimental.pallas import tpu as pltpu
from jax.experimental.pallas import tpu_sc as plsc

N = 10000
E = 320000
F_IN = 128
H = 64
R = 16
NC = 6

NCORES = 2
NSUB = 16
NW = NCORES * NSUB

LANES = 16
CW = 80
WIN_ROWS = 5
WIN_E = WIN_ROWS * CW
NWINDOWS = E // WIN_E
WPW = NWINDOWS // NW
WPS = NWINDOWS // NSUB

NR = N * R
CNT_STRIPE = NR // NSUB
ACC_STRIPE = N // NSUB
GR = 125
NG = N // GR
GPS = NG // NSUB

_mesh = plsc.VectorSubcoreMesh(core_axis_name="c", subcore_axis_name="s",
                               num_cores=NCORES, num_subcores=NSUB)

_sc_params = pltpu.CompilerParams(needs_layout_passes=False,
                                  use_tc_tiling_on_sc=False)


def _f32(*shape):
    return jax.ShapeDtypeStruct(shape, jnp.float32)


def _tca_body(x_ref, w2_ref, root_ref, b1_ref, xr_ref, ro_ref):
    xb = x_ref[...]
    xr_ref[...] = jnp.dot(xb, w2_ref[...], preferred_element_type=jnp.float32)
    ro_ref[...] = (jnp.dot(xb, root_ref[...], preferred_element_type=jnp.float32)
                   + b1_ref[...])


def _tc_a(x, w2, root, b1):
    bn = 1000
    grid = (N // bn,)
    return pl.pallas_call(
        _tca_body,
        grid=grid,
        in_specs=[
            pl.BlockSpec((bn, F_IN), lambda i: (i, 0)),
            pl.BlockSpec((F_IN, R * H), lambda i: (0, 0)),
            pl.BlockSpec((F_IN, H), lambda i: (0, 0)),
            pl.BlockSpec((1, H), lambda i: (0, 0)),
        ],
        out_specs=[
            pl.BlockSpec((bn, R * H), lambda i: (i, 0)),
            pl.BlockSpec((bn, H), lambda i: (i, 0)),
        ],
        out_shape=[_f32(N, R * H), _f32(N, H)],
    )(x, w2, root, b1)


def _sc1_body(xr_hbm, sde_hbm,
              p0_hbm, p1_hbm,
              cnt_sp, acc_sp,
              sdea0, sdea1, keya, sdew0, sdew1,
              gidx0, gidx1, keyw0, keyw1, invf0, invf1, rows0, rows1,
              onesv,
              semiA0, semiA1, semaA,
              semi0, semi1, semg0, semg1, semc0, semc1, sems0, sems1,
              semz):
    c = lax.axis_index("c")
    s = lax.axis_index("s")
    wid = c * NSUB + s

    z16 = jnp.zeros((LANES,), jnp.float32)
    o16 = jnp.ones((LANES,), jnp.float32)

    @pl.loop(0, GR)
    def _zrow(i):
        for q in range(H // LANES):
            rows0[i, pl.ds(q * LANES, LANES)] = z16

    @pl.loop(0, WIN_E // LANES)
    def _zinv(m):
        invf0[pl.ds(m * LANES, LANES)] = z16

    for q in range(CW // LANES):
        onesv[pl.ds(q * LANES, LANES)] = o16

    zcps = []
    for k in range(GPS):
        zcps.append(pltpu.async_copy(
            rows0.at[pl.ds(0, GR)],
            acc_sp.at[pl.ds(s * ACC_STRIPE + k * GR, GR)], semz))
    for m in range(CNT_STRIPE // WIN_E):
        zcps.append(pltpu.async_copy(
            invf0, cnt_sp.at[pl.ds(s * CNT_STRIPE + m * WIN_E, WIN_E)], semz))
    for cp in zcps:
        cp.wait()

    plsc.subcore_barrier()

    a_base = s * WPS

    def _hist_keys(sdea, kslot):
        for j in range(WIN_ROWS):
            for q in range(CW // LANES):
                sl = pl.ds(q * LANES, LANES)
                keya[kslot, j, sl] = (sdea[1, j, sl] * R + sdea[2, j, sl])

    def _hist_scat(kslot):
        for j in range(WIN_ROWS):
            pltpu.async_copy(onesv, cnt_sp.at[keya.at[kslot].at[j]],
                             semaA.at[kslot], add=True)

    def _hist_scat_wait(kslot):
        for j in range(WIN_ROWS):
            pltpu.make_async_copy(onesv, cnt_sp.at[keya.at[kslot].at[j]],
                                  semaA.at[kslot]).wait()

    pltpu.async_copy(sde_hbm.at[a_base], sdea0, semiA0)
    pltpu.async_copy(sde_hbm.at[a_base + 1], sdea1, semiA1)

    @pl.loop(0, WPS // 2)
    def _hist_iter(i):
        t0 = a_base + 2 * i
        for u, (sdea, semiA) in enumerate(((sdea0, semiA0),
                                           (sdea1, semiA1))):
            t = t0 + u
            pltpu.make_async_copy(sde_hbm.at[t], sdea, semiA).wait()

            @pl.when(i > 0)
            def _():
                _hist_scat_wait(u)
            _hist_keys(sdea, u)

            @pl.when(t + 2 < a_base + WPS)
            def _():
                pltpu.async_copy(sde_hbm.at[t + 2], sdea, semiA)
            _hist_scat(u)

    for kslot in range(2):
        _hist_scat_wait(kslot)

    plsc.subcore_barrier()

    b_base = wid * WPW
    bufs = ((sdew0, gidx0, keyw0, invf0, rows0, semi0, semg0, semc0, sems0),
            (sdew1, gidx1, keyw1, invf1, rows1, semi1, semg1, semc1, sems1))

    def _prep(b, t):
        sdew, gidx, keyw, invf, rows, semi, semg, semc = bufs[b][:8]
        pltpu.make_async_copy(sde_hbm.at[t], sdew, semi).wait()
        for j in range(WIN_ROWS):
            for q in range(CW // LANES):
                sl = pl.ds(q * LANES, LANES)
                e = sdew[2, j, sl]
                gidx[j, sl] = sdew[0, j, sl] * R + e
                keyw[j, sl] = sdew[1, j, sl] * R + e

        @pl.when(t + 2 < b_base + WPW)
        def _():
            pltpu.async_copy(sde_hbm.at[t + 2], sdew, semi)
        for j in range(WIN_ROWS):
            pltpu.async_copy(xr_hbm.at[gidx.at[j]],
                             rows.at[pl.ds(j * CW, CW)], semg)
        for j in range(WIN_ROWS):
            pltpu.async_copy(cnt_sp.at[keyw.at[j]],
                             invf.at[pl.ds(j * CW, CW)], semc)

    def _consume(b):
        sdew, gidx, keyw, invf, rows, _, semg, semc, sems = bufs[b]
        for j in range(WIN_ROWS):
            pltpu.make_async_copy(cnt_sp.at[keyw.at[j]],
                                  invf.at[pl.ds(j * CW, CW)], semc).wait()
        for j in range(WIN_ROWS):
            for q in range(CW // LANES):
                sl = pl.ds(q * LANES, LANES)
                fl = pl.ds(j * CW + q * LANES, LANES)
                invf[fl] = 1.0 / jnp.maximum(invf[fl], 1.0)
                keyw[j, sl] = keyw[j, sl] >> 4
        for j in range(WIN_ROWS):
            pltpu.make_async_copy(xr_hbm.at[gidx.at[j]],
                                  rows.at[pl.ds(j * CW, CW)], semg).wait()

        @plsc.parallel_loop(0, WIN_E, unroll=4)
        def _scale(e):
            nv = plsc.load_gather(invf, [jnp.broadcast_to(e, (LANES,))])
            for q in range(H // LANES):
                sl = pl.ds(q * LANES, LANES)
                rows[e, sl] = rows[e, sl] * nv

        for j in range(WIN_ROWS):
            pltpu.async_copy(rows.at[pl.ds(j * CW, CW)],
                             acc_sp.at[keyw.at[j]], sems, add=True)

    def _wait_scat(b):
        _, _, keyw, _, rows, _, _, _, sems = bufs[b]
        for j in range(WIN_ROWS):
            pltpu.make_async_copy(rows.at[pl.ds(j * CW, CW)],
                                  acc_sp.at[keyw.at[j]], sems).wait()

    pltpu.async_copy(sde_hbm.at[b_base], sdew0, semi0)
    pltpu.async_copy(sde_hbm.at[b_base + 1], sdew1, semi1)
    _prep(0, b_base)

    @pl.loop(0, WPW // 2)
    def _edge_iter(i):
        t = b_base + 2 * i
        _prep(1, t + 1)
        _consume(0)
        _wait_scat(0)
        _prep(0, t + 2)
        _consume(1)
        _wait_scat(1)

    _consume(0)
    _wait_scat(0)

    plsc.subcore_barrier()

    slots = [rows0.at[pl.ds(0, GR)], rows0.at[pl.ds(GR, GR)],
             rows0.at[pl.ds(2 * GR, GR)], rows1.at[pl.ds(0, GR)],
             rows1.at[pl.ds(GR, GR)]]
    for k in range(GPS):
        pltpu.sync_copy(acc_sp.at[pl.ds(s * ACC_STRIPE + k * GR, GR)],
                        slots[k])
    for k in range(GPS):
        rs = pl.ds(s * ACC_STRIPE + k * GR, GR)

        @pl.when(c == 0)
        def _():
            pltpu.async_copy(slots[k], p0_hbm.at[rs], semz)

        @pl.when(c == 1)
        def _():
            pltpu.async_copy(slots[k], p1_hbm.at[rs], semz)
    for k in range(GPS):
        rs = pl.ds(s * ACC_STRIPE + k * GR, GR)
        pltpu.make_async_copy(slots[k], p0_hbm.at[rs], semz).wait()


def _sc_1(xr2, sde):
    kern = pl.kernel(
        _sc1_body,
        out_type=[_f32(N, H), _f32(N, H)],
        mesh=_mesh,
        scratch_types=[
            pltpu.VMEM_SHARED((NR,), jnp.float32),
            pltpu.VMEM_SHARED((N, H), jnp.float32),
            pltpu.VMEM((3, WIN_ROWS, CW), jnp.int32),
            pltpu.VMEM((3, WIN_ROWS, CW), jnp.int32),
            pltpu.VMEM((2, WIN_ROWS, CW), jnp.int32),
            pltpu.VMEM((3, WIN_ROWS, CW), jnp.int32),
            pltpu.VMEM((3, WIN_ROWS, CW), jnp.int32),
            pltpu.VMEM((WIN_ROWS, CW), jnp.int32),
            pltpu.VMEM((WIN_ROWS, CW), jnp.int32),
            pltpu.VMEM((WIN_ROWS, CW), jnp.int32),
            pltpu.VMEM((WIN_ROWS, CW), jnp.int32),
            pltpu.VMEM((WIN_E,), jnp.float32),
            pltpu.VMEM((WIN_E,), jnp.float32),
            pltpu.VMEM((WIN_E, H), jnp.float32),
            pltpu.VMEM((WIN_E, H), jnp.float32),
            pltpu.VMEM((CW,), jnp.float32),
            pltpu.SemaphoreType.DMA,
            pltpu.SemaphoreType.DMA,
            pltpu.SemaphoreType.DMA((2,)),
            pltpu.SemaphoreType.DMA,
            pltpu.SemaphoreType.DMA,
            pltpu.SemaphoreType.DMA,
            pltpu.SemaphoreType.DMA,
            pltpu.SemaphoreType.DMA,
            pltpu.SemaphoreType.DMA,
            pltpu.SemaphoreType.DMA,
            pltpu.SemaphoreType.DMA,
            pltpu.SemaphoreType.DMA,
        ],
        compiler_params=_sc_params,
    )
    return kern(xr2, sde)


def _sc2_body(p0_hbm, p1_hbm, ro_hbm, sde_hbm,
              out1_hbm, q0_hbm, q1_hbm,
              acc_sp,
              sdew0, sdew1, srcw0, srcw1, dstw0, dstw1, rows0, rows1,
              semi0, semi1, semg0, semg1, sems0, sems1, semt, semz):
    c = lax.axis_index("c")
    s = lax.axis_index("s")
    wid = c * NSUB + s

    z16 = jnp.zeros((LANES,), jnp.float32)

    @pl.loop(0, GR)
    def _zrow(i):
        for q in range(H // LANES):
            rows0[i, pl.ds(q * LANES, LANES)] = z16

    zcps = [pltpu.async_copy(
        rows0.at[pl.ds(0, GR)],
        acc_sp.at[pl.ds(s * ACC_STRIPE + k * GR, GR)], semz)
        for k in range(GPS)]
    for cp in zcps:
        cp.wait()

    ta = rows0.at[pl.ds(0, GR)]
    tb = rows0.at[pl.ds(GR, GR)]
    tc_ = rows0.at[pl.ds(2 * GR, GR)]
    for k in range(GPS):
        g = pl.ds((s * GPS + k) * GR, GR)
        lcps = [pltpu.async_copy(p0_hbm.at[g], ta, semt),
                pltpu.async_copy(p1_hbm.at[g], tb, semt),
                pltpu.async_copy(ro_hbm.at[g], tc_, semt)]
        for cp in lcps:
            cp.wait()

        @pl.loop(0, GR)
        def _addrow(i):
            for q in range(H // LANES):
                sl = pl.ds(q * LANES, LANES)
                rows0[i, sl] = rows0[i, sl] + rows0[GR + i, sl] \
                    + rows0[2 * GR + i, sl]

        pltpu.sync_copy(ta, out1_hbm.at[g])

    plsc.subcore_barrier()

    b_base = wid * WPW
    bufs = ((sdew0, srcw0, dstw0, rows0, semi0, semg0, sems0),
            (sdew1, srcw1, dstw1, rows1, semi1, semg1, sems1))

    def _prep(b, t):
        sdew, srcw, dstw, rows, semi, semg, _ = bufs[b]
        pltpu.make_async_copy(sde_hbm.at[t], sdew, semi).wait()
        for j in range(WIN_ROWS):
            for q in range(CW // LANES):
                sl = pl.ds(q * LANES, LANES)
                srcw[j, sl] = sdew[0, j, sl]
                dstw[j, sl] = sdew[1, j, sl]

        @pl.when(t + 2 < b_base + WPW)
        def _():
            pltpu.async_copy(sde_hbm.at[t + 2], sdew, semi)
        for j in range(WIN_ROWS):
            pltpu.async_copy(out1_hbm.at[srcw.at[j]],
                             rows.at[pl.ds(j * CW, CW)], semg)

    def _consume(b):
        _, srcw, dstw, rows, _, semg, sems = bufs[b]
        for j in range(WIN_ROWS):
            pltpu.make_async_copy(out1_hbm.at[srcw.at[j]],
                                  rows.at[pl.ds(j * CW, CW)], semg).wait()
        for j in range(WIN_ROWS):
            pltpu.async_copy(rows.at[pl.ds(j * CW, CW)],
                             acc_sp.at[dstw.at[j]], sems, add=True)

    def _wait_scat(b):
        _, _, dstw, rows, _, _, sems = bufs[b]
        for j in range(WIN_ROWS):
            pltpu.make_async_copy(rows.at[pl.ds(j * CW, CW)],
                                  acc_sp.at[dstw.at[j]], sems).wait()

    pltpu.async_copy(sde_hbm.at[b_base], sdew0, semi0)
    pltpu.async_copy(sde_hbm.at[b_base + 1], sdew1, semi1)
    _prep(0, b_base)

    @pl.loop(0, WPW // 2)
    def _edge_iter(i):
        t = b_base + 2 * i
        _prep(1, t + 1)
        _consume(0)
        _wait_scat(0)
        _prep(0, t + 2)
        _consume(1)
        _wait_scat(1)

    _consume(0)
    _wait_scat(0)

    plsc.subcore_barrier()

    slots = [rows0.at[pl.ds(0, GR)], rows0.at[pl.ds(GR, GR)],
             rows0.at[pl.ds(2 * GR, GR)], rows1.at[pl.ds(0, GR)],
             rows1.at[pl.ds(GR, GR)]]
    for k in range(GPS):
        pltpu.sync_copy(acc_sp.at[pl.ds(s * ACC_STRIPE + k * GR, GR)],
                        slots[k])
    for k in range(GPS):
        rs = pl.ds(s * ACC_STRIPE + k * GR, GR)

        @pl.when(c == 0)
        def _():
            pltpu.async_copy(slots[k], q0_hbm.at[rs], semz)

        @pl.when(c == 1)
        def _():
            pltpu.async_copy(slots[k], q1_hbm.at[rs], semz)
    for k in range(GPS):
        rs = pl.ds(s * ACC_STRIPE + k * GR, GR)
        pltpu.make_async_copy(slots[k], q0_hbm.at[rs], semz).wait()


def _sc_2(p0, p1, ro, sde):
    kern = pl.kernel(
        _sc2_body,
        out_type=[_f32(N, H), _f32(N, H), _f32(N, H)],
        mesh=_mesh,
        scratch_types=[
            pltpu.VMEM_SHARED((N, H), jnp.float32),
            pltpu.VMEM((3, WIN_ROWS, CW), jnp.int32),
            pltpu.VMEM((3, WIN_ROWS, CW), jnp.int32),
            pltpu.VMEM((WIN_ROWS, CW), jnp.int32),
            pltpu.VMEM((WIN_ROWS, CW), jnp.int32),
            pltpu.VMEM((WIN_ROWS, CW), jnp.int32),
            pltpu.VMEM((WIN_ROWS, CW), jnp.int32),
            pltpu.VMEM((WIN_E, H), jnp.float32),
            pltpu.VMEM((WIN_E, H), jnp.float32),
            pltpu.SemaphoreType.DMA,
            pltpu.SemaphoreType.DMA,
            pltpu.SemaphoreType.DMA,
            pltpu.SemaphoreType.DMA,
            pltpu.SemaphoreType.DMA,
            pltpu.SemaphoreType.DMA,
            pltpu.SemaphoreType.DMA,
            pltpu.SemaphoreType.DMA,
        ],
        compiler_params=_sc_params,
    )
    return kern(p0, p1, ro, sde)


def _tcb_body(x_ref, o1_ref, q0_ref, q1_ref, wrel_ref, brel_ref, wr2_ref,
              wlx_ref, wlo_ref, bl_ref, ws_ref, bs_ref, out_ref):
    agg = q0_ref[...] + q1_ref[...]
    out1 = o1_ref[...]
    out2 = (jnp.dot(agg, wrel_ref[...], preferred_element_type=jnp.float32)
            + jnp.dot(out1, wr2_ref[...], preferred_element_type=jnp.float32)
            + brel_ref[...])
    h = jnp.maximum(
        jnp.dot(x_ref[...], wlx_ref[...], preferred_element_type=jnp.float32)
        + jnp.dot(out2, wlo_ref[...], preferred_element_type=jnp.float32)
        + bl_ref[...], 0.0)
    lg = jnp.dot(h, ws_ref[...], preferred_element_type=jnp.float32) + bs_ref[...]
    m = jnp.max(lg, axis=1, keepdims=True)
    sh = lg - m
    out_ref[...] = sh - jnp.log(jnp.sum(jnp.exp(sh), axis=1, keepdims=True))


def _tc_b(x, out1, q0, q1, wrel, brel, wr2, wlx, wlo, bl, ws, bs):
    bn = 1000
    grid = (N // bn,)

    def full(shape):
        return pl.BlockSpec(shape, lambda i: tuple(0 for _ in shape))

    return pl.pallas_call(
        _tcb_body,
        grid=grid,
        in_specs=[
            pl.BlockSpec((bn, F_IN), lambda i: (i, 0)),
            pl.BlockSpec((bn, H), lambda i: (i, 0)),
            pl.BlockSpec((bn, H), lambda i: (i, 0)),
            pl.BlockSpec((bn, H), lambda i: (i, 0)),
            full((H, H)),
            full((1, H)),
            full((H, H)),
            full((F_IN, H)),
            full((H, H)),
            full((1, H)),
            full((H, NC)),
            full((1, NC)),
        ],
        out_specs=pl.BlockSpec((bn, NC), lambda i: (i, 0)),
        out_shape=_f32(N, NC),
    )(x, out1, q0, q1, wrel, brel, wr2, wlx, wlo, bl, ws, bs)


def kernel(x, edge_index, edge_norm, edge_type, seq_lengths, umask,
           nodal_attn, avec, bases, comp, root, bias1, W_rel, b_rel,
           W_root2, Wl, bl, Ws, bs):
    w2 = jnp.einsum("rb,bio->iro", comp, bases).reshape(F_IN, R * H)
    src3 = edge_index[0].astype(jnp.int32).reshape(NWINDOWS, WIN_ROWS, CW)
    dst3 = edge_index[1].astype(jnp.int32).reshape(NWINDOWS, WIN_ROWS, CW)
    et3 = edge_type.astype(jnp.int32).reshape(NWINDOWS, WIN_ROWS, CW)
    sde = jnp.stack([src3, dst3, et3], axis=1)

    xr, root_out = _tc_a(x, w2, root, bias1.reshape(1, H))
    xr2 = xr.reshape(N * R, H)

    p0, p1 = _sc_1(xr2, sde)

    out1, q0, q1 = _sc_2(p0, p1, root_out, sde)

    return _tc_b(x, out1, q0, q1, W_rel,
                 b_rel.reshape(1, H), W_root2, Wl[:F_IN], Wl[F_IN:],
                 bl.reshape(1, H), Ws, bs.reshape(1, NC))


# speedup vs baseline: 42.8896x; 1.0226x over previous
import jax
import jax.numpy as jnp
from jax import lax
from jax.experimental import pallas as pl
from jax.experimental.pallas import tpu as pltpu
from jax.experimental.pallas import tpu_sc as plsc

N = 10000
E = 320000
F_IN = 128
H = 64
R = 16
NC = 6

NCORES = 2
NSUB = 16
NW = NCORES * NSUB

LANES = 16
CW = 80
WIN_ROWS = 5
WIN_E = WIN_ROWS * CW
NWINDOWS = E // WIN_E
WPW = NWINDOWS // NW
WPS = NWINDOWS // NSUB

NR = N * R
CNT_STRIPE = NR // NSUB
ACC_STRIPE = N // NSUB
GR = 125
NG = N // GR
GPS = NG // NSUB

_mesh = plsc.VectorSubcoreMesh(core_axis_name="c", subcore_axis_name="s",
                               num_cores=NCORES, num_subcores=NSUB)

_sc_params = pltpu.CompilerParams(needs_layout_passes=False,
                                  use_tc_tiling_on_sc=False)


def _f32(*shape):
    return jax.ShapeDtypeStruct(shape, jnp.float32)


def _tca_body(x_ref, w2_ref, root_ref, b1_ref, xr_ref, ro_ref):
    xb = x_ref[...]
    xr_ref[...] = jnp.dot(xb, w2_ref[...], preferred_element_type=jnp.float32)
    ro_ref[...] = (jnp.dot(xb, root_ref[...], preferred_element_type=jnp.float32)
                   + b1_ref[...])


def _tc_a(x, w2, root, b1):
    bn = 1000
    grid = (N // bn,)
    return pl.pallas_call(
        _tca_body,
        grid=grid,
        in_specs=[
            pl.BlockSpec((bn, F_IN), lambda i: (i, 0)),
            pl.BlockSpec((F_IN, R * H), lambda i: (0, 0)),
            pl.BlockSpec((F_IN, H), lambda i: (0, 0)),
            pl.BlockSpec((1, H), lambda i: (0, 0)),
        ],
        out_specs=[
            pl.BlockSpec((bn, R * H), lambda i: (i, 0)),
            pl.BlockSpec((bn, H), lambda i: (i, 0)),
        ],
        out_shape=[_f32(N, R * H), _f32(N, H)],
    )(x, w2, root, b1)


def _sc1_body(xr_hbm, sde_hbm,
              p0_hbm, p1_hbm,
              cnt_sp, acc_sp,
              sdea0, sdea1, keya, sdew0, sdew1,
              gidx0, gidx1, keyw0, keyw1, invf0, invf1, rows0, rows1,
              onesv,
              semiA0, semiA1, semaA,
              semi0, semi1, semg0, semg1, semc0, semc1, sems0, sems1,
              semz):
    c = lax.axis_index("c")
    s = lax.axis_index("s")
    wid = c * NSUB + s

    z16 = jnp.zeros((LANES,), jnp.float32)
    o16 = jnp.ones((LANES,), jnp.float32)

    @pl.loop(0, GR)
    def _zrow(i):
        for q in range(H // LANES):
            rows0[i, pl.ds(q * LANES, LANES)] = z16

    @pl.loop(0, WIN_E // LANES)
    def _zinv(m):
        invf0[pl.ds(m * LANES, LANES)] = z16

    for q in range(CW // LANES):
        onesv[pl.ds(q * LANES, LANES)] = o16

    zcps = []
    for k in range(GPS):
        zcps.append(pltpu.async_copy(
            rows0.at[pl.ds(0, GR)],
            acc_sp.at[pl.ds(s * ACC_STRIPE + k * GR, GR)], semz))
    for m in range(CNT_STRIPE // WIN_E):
        zcps.append(pltpu.async_copy(
            invf0, cnt_sp.at[pl.ds(s * CNT_STRIPE + m * WIN_E, WIN_E)], semz))
    for cp in zcps:
        cp.wait()

    plsc.subcore_barrier()

    a_base = s * WPS

    def _hist_keys(sdea, kslot):
        for j in range(WIN_ROWS):
            for q in range(CW // LANES):
                sl = pl.ds(q * LANES, LANES)
                keya[kslot, j, sl] = (sdea[1, j, sl] * R + sdea[2, j, sl])

    def _hist_scat(kslot):
        for j in range(WIN_ROWS):
            pltpu.async_copy(onesv, cnt_sp.at[keya.at[kslot].at[j]],
                             semaA.at[kslot], add=True)

    def _hist_scat_wait(kslot):
        for j in range(WIN_ROWS):
            pltpu.make_async_copy(onesv, cnt_sp.at[keya.at[kslot].at[j]],
                                  semaA.at[kslot]).wait()

    pltpu.async_copy(sde_hbm.at[a_base], sdea0, semiA0)
    pltpu.async_copy(sde_hbm.at[a_base + 1], sdea1, semiA1)

    @pl.loop(0, WPS // 2)
    def _hist_iter(i):
        t0 = a_base + 2 * i
        for u, (sdea, semiA) in enumerate(((sdea0, semiA0),
                                           (sdea1, semiA1))):
            t = t0 + u
            pltpu.make_async_copy(sde_hbm.at[t], sdea, semiA).wait()

            @pl.when(i > 0)
            def _():
                _hist_scat_wait(u)
            _hist_keys(sdea, u)

            @pl.when(t + 2 < a_base + WPS)
            def _():
                pltpu.async_copy(sde_hbm.at[t + 2], sdea, semiA)
            _hist_scat(u)

    for kslot in range(2):
        _hist_scat_wait(kslot)

    plsc.subcore_barrier()

    b_base = wid * WPW
    bufs = ((sdew0, gidx0, keyw0, invf0, rows0, semi0, semg0, semc0, sems0),
            (sdew1, gidx1, keyw1, invf1, rows1, semi1, semg1, semc1, sems1))

    def _prep(b, t):
        sdew, gidx, keyw, invf, rows, semi, semg, semc = bufs[b][:8]
        pltpu.make_async_copy(sde_hbm.at[t], sdew, semi).wait()
        for j in range(WIN_ROWS):
            for q in range(CW // LANES):
                sl = pl.ds(q * LANES, LANES)
                e = sdew[2, j, sl]
                gidx[j, sl] = sdew[0, j, sl] * R + e
                keyw[j, sl] = sdew[1, j, sl] * R + e

        @pl.when(t + 2 < b_base + WPW)
        def _():
            pltpu.async_copy(sde_hbm.at[t + 2], sdew, semi)
        for j in range(WIN_ROWS):
            pltpu.async_copy(xr_hbm.at[gidx.at[j]],
                             rows.at[pl.ds(j * CW, CW)], semg)
        for j in range(WIN_ROWS):
            pltpu.async_copy(cnt_sp.at[keyw.at[j]],
                             invf.at[pl.ds(j * CW, CW)], semc)

    def _consume(b):
        sdew, gidx, keyw, invf, rows, _, semg, semc, sems = bufs[b]
        for j in range(WIN_ROWS):
            pltpu.make_async_copy(cnt_sp.at[keyw.at[j]],
                                  invf.at[pl.ds(j * CW, CW)], semc).wait()
        for j in range(WIN_ROWS):
            for q in range(CW // LANES):
                sl = pl.ds(q * LANES, LANES)
                fl = pl.ds(j * CW + q * LANES, LANES)
                invf[fl] = 1.0 / jnp.maximum(invf[fl], 1.0)
                keyw[j, sl] = keyw[j, sl] >> 4
        for j in range(WIN_ROWS):
            pltpu.make_async_copy(xr_hbm.at[gidx.at[j]],
                                  rows.at[pl.ds(j * CW, CW)], semg).wait()

            @plsc.parallel_loop(j * CW, (j + 1) * CW, unroll=4)
            def _scale(e):
                nv = plsc.load_gather(invf, [jnp.broadcast_to(e, (LANES,))])
                for q in range(H // LANES):
                    sl = pl.ds(q * LANES, LANES)
                    rows[e, sl] = rows[e, sl] * nv

            pltpu.async_copy(rows.at[pl.ds(j * CW, CW)],
                             acc_sp.at[keyw.at[j]], sems, add=True)

    def _wait_scat(b):
        _, _, keyw, _, rows, _, _, _, sems = bufs[b]
        for j in range(WIN_ROWS):
            pltpu.make_async_copy(rows.at[pl.ds(j * CW, CW)],
                                  acc_sp.at[keyw.at[j]], sems).wait()

    pltpu.async_copy(sde_hbm.at[b_base], sdew0, semi0)
    pltpu.async_copy(sde_hbm.at[b_base + 1], sdew1, semi1)
    _prep(0, b_base)

    @pl.loop(0, WPW // 2)
    def _edge_iter(i):
        t = b_base + 2 * i
        _prep(1, t + 1)
        _consume(0)
        _wait_scat(0)
        _prep(0, t + 2)
        _consume(1)
        _wait_scat(1)

    _consume(0)
    _wait_scat(0)

    plsc.subcore_barrier()

    slots = [rows0.at[pl.ds(0, GR)], rows0.at[pl.ds(GR, GR)],
             rows0.at[pl.ds(2 * GR, GR)], rows1.at[pl.ds(0, GR)],
             rows1.at[pl.ds(GR, GR)]]
    for k in range(GPS):
        pltpu.sync_copy(acc_sp.at[pl.ds(s * ACC_STRIPE + k * GR, GR)],
                        slots[k])
    for k in range(GPS):
        rs = pl.ds(s * ACC_STRIPE + k * GR, GR)

        @pl.when(c == 0)
        def _():
            pltpu.async_copy(slots[k], p0_hbm.at[rs], semz)

        @pl.when(c == 1)
        def _():
            pltpu.async_copy(slots[k], p1_hbm.at[rs], semz)
    for k in range(GPS):
        rs = pl.ds(s * ACC_STRIPE + k * GR, GR)
        pltpu.make_async_copy(slots[k], p0_hbm.at[rs], semz).wait()


def _sc_1(xr2, sde):
    kern = pl.kernel(
        _sc1_body,
        out_type=[_f32(N, H), _f32(N, H)],
        mesh=_mesh,
        scratch_types=[
            pltpu.VMEM_SHARED((NR,), jnp.float32),
            pltpu.VMEM_SHARED((N, H), jnp.float32),
            pltpu.VMEM((3, WIN_ROWS, CW), jnp.int32),
            pltpu.VMEM((3, WIN_ROWS, CW), jnp.int32),
            pltpu.VMEM((2, WIN_ROWS, CW), jnp.int32),
            pltpu.VMEM((3, WIN_ROWS, CW), jnp.int32),
            pltpu.VMEM((3, WIN_ROWS, CW), jnp.int32),
            pltpu.VMEM((WIN_ROWS, CW), jnp.int32),
            pltpu.VMEM((WIN_ROWS, CW), jnp.int32),
            pltpu.VMEM((WIN_ROWS, CW), jnp.int32),
            pltpu.VMEM((WIN_ROWS, CW), jnp.int32),
            pltpu.VMEM((WIN_E,), jnp.float32),
            pltpu.VMEM((WIN_E,), jnp.float32),
            pltpu.VMEM((WIN_E, H), jnp.float32),
            pltpu.VMEM((WIN_E, H), jnp.float32),
            pltpu.VMEM((CW,), jnp.float32),
            pltpu.SemaphoreType.DMA,
            pltpu.SemaphoreType.DMA,
            pltpu.SemaphoreType.DMA((2,)),
            pltpu.SemaphoreType.DMA,
            pltpu.SemaphoreType.DMA,
            pltpu.SemaphoreType.DMA,
            pltpu.SemaphoreType.DMA,
            pltpu.SemaphoreType.DMA,
            pltpu.SemaphoreType.DMA,
            pltpu.SemaphoreType.DMA,
            pltpu.SemaphoreType.DMA,
            pltpu.SemaphoreType.DMA,
        ],
        compiler_params=_sc_params,
    )
    return kern(xr2, sde)


def _sc2_body(p0_hbm, p1_hbm, ro_hbm, sde_hbm,
              out1_hbm, q0_hbm, q1_hbm,
              acc_sp,
              sdew0, sdew1, srcw0, srcw1, dstw0, dstw1, rows0, rows1,
              semi0, semi1, semg0, semg1, sems0, sems1, semt, semz):
    c = lax.axis_index("c")
    s = lax.axis_index("s")
    wid = c * NSUB + s

    z16 = jnp.zeros((LANES,), jnp.float32)

    @pl.loop(0, GR)
    def _zrow(i):
        for q in range(H // LANES):
            rows0[i, pl.ds(q * LANES, LANES)] = z16

    zcps = [pltpu.async_copy(
        rows0.at[pl.ds(0, GR)],
        acc_sp.at[pl.ds(s * ACC_STRIPE + k * GR, GR)], semz)
        for k in range(GPS)]
    for cp in zcps:
        cp.wait()

    ta = rows0.at[pl.ds(0, GR)]
    tb = rows0.at[pl.ds(GR, GR)]
    tc_ = rows0.at[pl.ds(2 * GR, GR)]
    for k in range(GPS):
        g = pl.ds((s * GPS + k) * GR, GR)
        lcps = [pltpu.async_copy(p0_hbm.at[g], ta, semt),
                pltpu.async_copy(p1_hbm.at[g], tb, semt),
                pltpu.async_copy(ro_hbm.at[g], tc_, semt)]
        for cp in lcps:
            cp.wait()

        @pl.loop(0, GR)
        def _addrow(i):
            for q in range(H // LANES):
                sl = pl.ds(q * LANES, LANES)
                rows0[i, sl] = rows0[i, sl] + rows0[GR + i, sl] \
                    + rows0[2 * GR + i, sl]

        pltpu.sync_copy(ta, out1_hbm.at[g])

    plsc.subcore_barrier()

    b_base = wid * WPW
    bufs = ((sdew0, srcw0, dstw0, rows0, semi0, semg0, sems0),
            (sdew1, srcw1, dstw1, rows1, semi1, semg1, sems1))

    def _prep(b, t):
        sdew, srcw, dstw, rows, semi, semg, _ = bufs[b]
        pltpu.make_async_copy(sde_hbm.at[t], sdew, semi).wait()
        for j in range(WIN_ROWS):
            for q in range(CW // LANES):
                sl = pl.ds(q * LANES, LANES)
                srcw[j, sl] = sdew[0, j, sl]
                dstw[j, sl] = sdew[1, j, sl]

        @pl.when(t + 2 < b_base + WPW)
        def _():
            pltpu.async_copy(sde_hbm.at[t + 2], sdew, semi)
        for j in range(WIN_ROWS):
            pltpu.async_copy(out1_hbm.at[srcw.at[j]],
                             rows.at[pl.ds(j * CW, CW)], semg)

    def _consume(b):
        _, srcw, dstw, rows, _, semg, sems = bufs[b]
        for j in range(WIN_ROWS):
            pltpu.make_async_copy(out1_hbm.at[srcw.at[j]],
                                  rows.at[pl.ds(j * CW, CW)], semg).wait()
        for j in range(WIN_ROWS):
            pltpu.async_copy(rows.at[pl.ds(j * CW, CW)],
                             acc_sp.at[dstw.at[j]], sems, add=True)

    def _wait_scat(b):
        _, _, dstw, rows, _, _, sems = bufs[b]
        for j in range(WIN_ROWS):
            pltpu.make_async_copy(rows.at[pl.ds(j * CW, CW)],
                                  acc_sp.at[dstw.at[j]], sems).wait()

    pltpu.async_copy(sde_hbm.at[b_base], sdew0, semi0)
    pltpu.async_copy(sde_hbm.at[b_base + 1], sdew1, semi1)
    _prep(0, b_base)

    @pl.loop(0, WPW // 2)
    def _edge_iter(i):
        t = b_base + 2 * i
        _prep(1, t + 1)
        _consume(0)
        _wait_scat(0)
        _prep(0, t + 2)
        _consume(1)
        _wait_scat(1)

    _consume(0)
    _wait_scat(0)

    plsc.subcore_barrier()

    slots = [rows0.at[pl.ds(0, GR)], rows0.at[pl.ds(GR, GR)],
             rows0.at[pl.ds(2 * GR, GR)], rows1.at[pl.ds(0, GR)],
             rows1.at[pl.ds(GR, GR)]]
    for k in range(GPS):
        pltpu.sync_copy(acc_sp.at[pl.ds(s * ACC_STRIPE + k * GR, GR)],
                        slots[k])
    for k in range(GPS):
        rs = pl.ds(s * ACC_STRIPE + k * GR, GR)

        @pl.when(c == 0)
        def _():
            pltpu.async_copy(slots[k], q0_hbm.at[rs], semz)

        @pl.when(c == 1)
        def _():
            pltpu.async_copy(slots[k], q1_hbm.at[rs], semz)
    for k in range(GPS):
        rs = pl.ds(s * ACC_STRIPE + k * GR, GR)
        pltpu.make_async_copy(slots[k], q0_hbm.at[rs], semz).wait()


def _sc_2(p0, p1, ro, sde):
    kern = pl.kernel(
        _sc2_body,
        out_type=[_f32(N, H), _f32(N, H), _f32(N, H)],
        mesh=_mesh,
        scratch_types=[
            pltpu.VMEM_SHARED((N, H), jnp.float32),
            pltpu.VMEM((3, WIN_ROWS, CW), jnp.int32),
            pltpu.VMEM((3, WIN_ROWS, CW), jnp.int32),
            pltpu.VMEM((WIN_ROWS, CW), jnp.int32),
            pltpu.VMEM((WIN_ROWS, CW), jnp.int32),
            pltpu.VMEM((WIN_ROWS, CW), jnp.int32),
            pltpu.VMEM((WIN_ROWS, CW), jnp.int32),
            pltpu.VMEM((WIN_E, H), jnp.float32),
            pltpu.VMEM((WIN_E, H), jnp.float32),
            pltpu.SemaphoreType.DMA,
            pltpu.SemaphoreType.DMA,
            pltpu.SemaphoreType.DMA,
            pltpu.SemaphoreType.DMA,
            pltpu.SemaphoreType.DMA,
            pltpu.SemaphoreType.DMA,
            pltpu.SemaphoreType.DMA,
            pltpu.SemaphoreType.DMA,
        ],
        compiler_params=_sc_params,
    )
    return kern(p0, p1, ro, sde)


def _tcb_body(x_ref, o1_ref, q0_ref, q1_ref, wrel_ref, brel_ref, wr2_ref,
              wlx_ref, wlo_ref, bl_ref, ws_ref, bs_ref, out_ref):
    agg = q0_ref[...] + q1_ref[...]
    out1 = o1_ref[...]
    out2 = (jnp.dot(agg, wrel_ref[...], preferred_element_type=jnp.float32)
            + jnp.dot(out1, wr2_ref[...], preferred_element_type=jnp.float32)
            + brel_ref[...])
    h = jnp.maximum(
        jnp.dot(x_ref[...], wlx_ref[...], preferred_element_type=jnp.float32)
        + jnp.dot(out2, wlo_ref[...], preferred_element_type=jnp.float32)
        + bl_ref[...], 0.0)
    lg = jnp.dot(h, ws_ref[...], preferred_element_type=jnp.float32) + bs_ref[...]
    m = jnp.max(lg, axis=1, keepdims=True)
    sh = lg - m
    out_ref[...] = sh - jnp.log(jnp.sum(jnp.exp(sh), axis=1, keepdims=True))


def _tc_b(x, out1, q0, q1, wrel, brel, wr2, wlx, wlo, bl, ws, bs):
    bn = 1000
    grid = (N // bn,)

    def full(shape):
        return pl.BlockSpec(shape, lambda i: tuple(0 for _ in shape))

    return pl.pallas_call(
        _tcb_body,
        grid=grid,
        in_specs=[
            pl.BlockSpec((bn, F_IN), lambda i: (i, 0)),
            pl.BlockSpec((bn, H), lambda i: (i, 0)),
            pl.BlockSpec((bn, H), lambda i: (i, 0)),
            pl.BlockSpec((bn, H), lambda i: (i, 0)),
            full((H, H)),
            full((1, H)),
            full((H, H)),
            full((F_IN, H)),
            full((H, H)),
            full((1, H)),
            full((H, NC)),
            full((1, NC)),
        ],
        out_specs=pl.BlockSpec((bn, NC), lambda i: (i, 0)),
        out_shape=_f32(N, NC),
    )(x, out1, q0, q1, wrel, brel, wr2, wlx, wlo, bl, ws, bs)


def kernel(x, edge_index, edge_norm, edge_type, seq_lengths, umask,
           nodal_attn, avec, bases, comp, root, bias1, W_rel, b_rel,
           W_root2, Wl, bl, Ws, bs):
    w2 = jnp.einsum("rb,bio->iro", comp, bases).reshape(F_IN, R * H)
    src3 = edge_index[0].astype(jnp.int32).reshape(NWINDOWS, WIN_ROWS, CW)
    dst3 = edge_index[1].astype(jnp.int32).reshape(NWINDOWS, WIN_ROWS, CW)
    et3 = edge_type.astype(jnp.int32).reshape(NWINDOWS, WIN_ROWS, CW)
    sde = jnp.stack([src3, dst3, et3], axis=1)

    xr, root_out = _tc_a(x, w2, root, bias1.reshape(1, H))
    xr2 = xr.reshape(N * R, H)

    p0, p1 = _sc_1(xr2, sde)

    out1, q0, q1 = _sc_2(p0, p1, root_out, sde)

    return _tc_b(x, out1, q0, q1, W_rel,
                 b_rel.reshape(1, H), W_root2, Wl[:F_IN], Wl[F_IN:],
                 bl.reshape(1, H), Ws, bs.reshape(1, NC))


# speedup vs baseline: 43.3556x; 1.0109x over previous
import jax
import jax.numpy as jnp
from jax import lax
from jax.experimental import pallas as pl
from jax.experimental.pallas import tpu as pltpu
from jax.experimental.pallas import tpu_sc as plsc

N = 10000
E = 320000
F_IN = 128
H = 64
R = 16
NC = 6

NCORES = 2
NSUB = 16
NW = NCORES * NSUB

LANES = 16
CW = 80
WIN_ROWS = 5
WIN_E = WIN_ROWS * CW
NWINDOWS = E // WIN_E
WPW = NWINDOWS // NW
WPS = NWINDOWS // NSUB

NR = N * R
CNT_STRIPE = NR // NSUB
ACC_STRIPE = N // NSUB
GR = 125
NG = N // GR
GPS = NG // NSUB

_mesh = plsc.VectorSubcoreMesh(core_axis_name="c", subcore_axis_name="s",
                               num_cores=NCORES, num_subcores=NSUB)

_sc_params = pltpu.CompilerParams(needs_layout_passes=False,
                                  use_tc_tiling_on_sc=False)


def _f32(*shape):
    return jax.ShapeDtypeStruct(shape, jnp.float32)


def _tca_body(x_ref, w2_ref, root_ref, b1_ref, xr_ref, ro_ref):
    xb = x_ref[...]
    xr_ref[...] = jnp.dot(xb, w2_ref[...], preferred_element_type=jnp.float32)
    ro_ref[...] = (jnp.dot(xb, root_ref[...], preferred_element_type=jnp.float32)
                   + b1_ref[...])


def _tc_a(x, w2, root, b1):
    bn = 1000
    grid = (N // bn,)
    return pl.pallas_call(
        _tca_body,
        grid=grid,
        in_specs=[
            pl.BlockSpec((bn, F_IN), lambda i: (i, 0)),
            pl.BlockSpec((F_IN, R * H), lambda i: (0, 0)),
            pl.BlockSpec((F_IN, H), lambda i: (0, 0)),
            pl.BlockSpec((1, H), lambda i: (0, 0)),
        ],
        out_specs=[
            pl.BlockSpec((bn, R * H), lambda i: (i, 0)),
            pl.BlockSpec((bn, H), lambda i: (i, 0)),
        ],
        out_shape=[_f32(N, R * H), _f32(N, H)],
    )(x, w2, root, b1)


def _sc1_body(xr_hbm, sde_hbm,
              p0_hbm, p1_hbm,
              cnt_sp, acc_sp,
              sdea0, sdea1, keya, sdew0, sdew1,
              gidx0, gidx1, keyw0, keyw1, invf0, invf1, rows0, rows1,
              onesv,
              semiA0, semiA1, semaA,
              semi0, semi1, semg0, semg1, semc0, semc1, sems0, sems1,
              semz):
    c = lax.axis_index("c")
    s = lax.axis_index("s")
    wid = c * NSUB + s

    z16 = jnp.zeros((LANES,), jnp.float32)
    o16 = jnp.ones((LANES,), jnp.float32)

    @pl.loop(0, GR)
    def _zrow(i):
        for q in range(H // LANES):
            rows0[i, pl.ds(q * LANES, LANES)] = z16

    @pl.loop(0, WIN_E // LANES)
    def _zinv(m):
        invf0[pl.ds(m * LANES, LANES)] = z16

    for q in range(CW // LANES):
        onesv[pl.ds(q * LANES, LANES)] = o16

    zcps = []
    for k in range(GPS):
        zcps.append(pltpu.async_copy(
            rows0.at[pl.ds(0, GR)],
            acc_sp.at[pl.ds(s * ACC_STRIPE + k * GR, GR)], semz))
    for m in range(CNT_STRIPE // WIN_E):
        zcps.append(pltpu.async_copy(
            invf0, cnt_sp.at[pl.ds(s * CNT_STRIPE + m * WIN_E, WIN_E)], semz))
    for cp in zcps:
        cp.wait()

    plsc.subcore_barrier()

    a_base = s * WPS

    def _hist_keys(sdea, kslot):
        for j in range(WIN_ROWS):
            for q in range(CW // LANES):
                sl = pl.ds(q * LANES, LANES)
                keya[kslot, j, sl] = (sdea[1, j, sl] * R + sdea[2, j, sl])

    def _hist_scat(kslot):
        for j in range(WIN_ROWS):
            pltpu.async_copy(onesv, cnt_sp.at[keya.at[kslot].at[j]],
                             semaA.at[kslot], add=True)

    def _hist_scat_wait(kslot):
        for j in range(WIN_ROWS):
            pltpu.make_async_copy(onesv, cnt_sp.at[keya.at[kslot].at[j]],
                                  semaA.at[kslot]).wait()

    pltpu.async_copy(sde_hbm.at[a_base], sdea0, semiA0)
    pltpu.async_copy(sde_hbm.at[a_base + 1], sdea1, semiA1)

    @pl.loop(0, WPS // 2)
    def _hist_iter(i):
        t0 = a_base + 2 * i
        for u, (sdea, semiA) in enumerate(((sdea0, semiA0),
                                           (sdea1, semiA1))):
            t = t0 + u
            pltpu.make_async_copy(sde_hbm.at[t], sdea, semiA).wait()

            @pl.when(i > 0)
            def _():
                _hist_scat_wait(u)
            _hist_keys(sdea, u)

            @pl.when(t + 2 < a_base + WPS)
            def _():
                pltpu.async_copy(sde_hbm.at[t + 2], sdea, semiA)
            _hist_scat(u)

    for kslot in range(2):
        _hist_scat_wait(kslot)

    plsc.subcore_barrier()

    b_base = wid * WPW
    bufs = ((sdew0, gidx0, keyw0, invf0, rows0, semi0, semg0, semc0, sems0),
            (sdew1, gidx1, keyw1, invf1, rows1, semi1, semg1, semc1, sems1))

    def _prep(b, t):
        sdew, gidx, keyw, invf, rows, semi, semg, semc = bufs[b][:8]
        pltpu.make_async_copy(sde_hbm.at[t], sdew, semi).wait()
        for j in range(WIN_ROWS):
            for q in range(CW // LANES):
                sl = pl.ds(q * LANES, LANES)
                e = sdew[2, j, sl]
                gidx[j, sl] = sdew[0, j, sl] * R + e
                keyw[j, sl] = sdew[1, j, sl] * R + e

        @pl.when(t + 2 < b_base + WPW)
        def _():
            pltpu.async_copy(sde_hbm.at[t + 2], sdew, semi)
        for j in range(WIN_ROWS):
            pltpu.async_copy(xr_hbm.at[gidx.at[j]],
                             rows.at[pl.ds(j * CW, CW)], semg)
        for j in range(WIN_ROWS):
            pltpu.async_copy(cnt_sp.at[keyw.at[j]],
                             invf.at[pl.ds(j * CW, CW)], semc)

    def _consume(b):
        sdew, gidx, keyw, invf, rows, _, semg, semc, sems = bufs[b]
        for j in range(WIN_ROWS):
            pltpu.make_async_copy(cnt_sp.at[keyw.at[j]],
                                  invf.at[pl.ds(j * CW, CW)], semc).wait()
        for j in range(WIN_ROWS):
            for q in range(CW // LANES):
                sl = pl.ds(q * LANES, LANES)
                fl = pl.ds(j * CW + q * LANES, LANES)
                invf[fl] = 1.0 / jnp.maximum(invf[fl], 1.0)
                keyw[j, sl] = keyw[j, sl] >> 4
        for j in range(WIN_ROWS):
            pltpu.make_async_copy(xr_hbm.at[gidx.at[j]],
                                  rows.at[pl.ds(j * CW, CW)], semg).wait()

            @plsc.parallel_loop(j * CW, (j + 1) * CW, unroll=4)
            def _scale(e):
                nv = plsc.load_gather(invf, [jnp.broadcast_to(e, (LANES,))])
                for q in range(H // LANES):
                    sl = pl.ds(q * LANES, LANES)
                    rows[e, sl] = rows[e, sl] * nv

            pltpu.async_copy(rows.at[pl.ds(j * CW, CW)],
                             acc_sp.at[keyw.at[j]], sems, add=True)

    def _wait_scat(b):
        _, _, keyw, _, rows, _, _, _, sems = bufs[b]
        for j in range(WIN_ROWS):
            pltpu.make_async_copy(rows.at[pl.ds(j * CW, CW)],
                                  acc_sp.at[keyw.at[j]], sems).wait()

    pltpu.async_copy(sde_hbm.at[b_base], sdew0, semi0)
    pltpu.async_copy(sde_hbm.at[b_base + 1], sdew1, semi1)
    _prep(0, b_base)

    @pl.loop(0, WPW // 2)
    def _edge_iter(i):
        t = b_base + 2 * i
        _prep(1, t + 1)
        _consume(0)
        _wait_scat(0)
        _prep(0, t + 2)
        _consume(1)
        _wait_scat(1)

    _consume(0)
    _wait_scat(0)

    plsc.subcore_barrier()

    slots = [rows0.at[pl.ds(0, GR)], rows0.at[pl.ds(GR, GR)],
             rows0.at[pl.ds(2 * GR, GR)], rows1.at[pl.ds(0, GR)],
             rows1.at[pl.ds(GR, GR)]]
    for k in range(GPS):
        pltpu.sync_copy(acc_sp.at[pl.ds(s * ACC_STRIPE + k * GR, GR)],
                        slots[k])
    for k in range(GPS):
        rs = pl.ds(s * ACC_STRIPE + k * GR, GR)

        @pl.when(c == 0)
        def _():
            pltpu.async_copy(slots[k], p0_hbm.at[rs], semz)

        @pl.when(c == 1)
        def _():
            pltpu.async_copy(slots[k], p1_hbm.at[rs], semz)
    for k in range(GPS):
        rs = pl.ds(s * ACC_STRIPE + k * GR, GR)
        pltpu.make_async_copy(slots[k], p0_hbm.at[rs], semz).wait()


def _sc_1(xr2, sde):
    kern = pl.kernel(
        _sc1_body,
        out_type=[_f32(N, H), _f32(N, H)],
        mesh=_mesh,
        scratch_types=[
            pltpu.VMEM_SHARED((NR,), jnp.float32),
            pltpu.VMEM_SHARED((N, H), jnp.float32),
            pltpu.VMEM((3, WIN_ROWS, CW), jnp.int32),
            pltpu.VMEM((3, WIN_ROWS, CW), jnp.int32),
            pltpu.VMEM((2, WIN_ROWS, CW), jnp.int32),
            pltpu.VMEM((3, WIN_ROWS, CW), jnp.int32),
            pltpu.VMEM((3, WIN_ROWS, CW), jnp.int32),
            pltpu.VMEM((WIN_ROWS, CW), jnp.int32),
            pltpu.VMEM((WIN_ROWS, CW), jnp.int32),
            pltpu.VMEM((WIN_ROWS, CW), jnp.int32),
            pltpu.VMEM((WIN_ROWS, CW), jnp.int32),
            pltpu.VMEM((WIN_E,), jnp.float32),
            pltpu.VMEM((WIN_E,), jnp.float32),
            pltpu.VMEM((WIN_E, H), jnp.float32),
            pltpu.VMEM((WIN_E, H), jnp.float32),
            pltpu.VMEM((CW,), jnp.float32),
            pltpu.SemaphoreType.DMA,
            pltpu.SemaphoreType.DMA,
            pltpu.SemaphoreType.DMA((2,)),
            pltpu.SemaphoreType.DMA,
            pltpu.SemaphoreType.DMA,
            pltpu.SemaphoreType.DMA,
            pltpu.SemaphoreType.DMA,
            pltpu.SemaphoreType.DMA,
            pltpu.SemaphoreType.DMA,
            pltpu.SemaphoreType.DMA,
            pltpu.SemaphoreType.DMA,
            pltpu.SemaphoreType.DMA,
        ],
        compiler_params=_sc_params,
    )
    return kern(xr2, sde)


def _sc2_body(p0_hbm, p1_hbm, ro_hbm, sde_hbm,
              out1_hbm, q0_hbm, q1_hbm,
              acc_sp,
              sdew0, sdew1, srcw0, srcw1, dstw0, dstw1, rows0, rows1,
              semi0, semi1, semg0, semg1, sems0, sems1, semt, semz):
    c = lax.axis_index("c")
    s = lax.axis_index("s")
    wid = c * NSUB + s

    z16 = jnp.zeros((LANES,), jnp.float32)

    @pl.loop(0, GR)
    def _zrow(i):
        for q in range(H // LANES):
            rows0[i, pl.ds(q * LANES, LANES)] = z16

    zcps = [pltpu.async_copy(
        rows0.at[pl.ds(0, GR)],
        acc_sp.at[pl.ds(s * ACC_STRIPE + k * GR, GR)], semz)
        for k in range(GPS)]
    for cp in zcps:
        cp.wait()

    ta = rows0.at[pl.ds(0, GR)]
    tb = rows0.at[pl.ds(GR, GR)]
    tc_ = rows0.at[pl.ds(2 * GR, GR)]
    for k in range(GPS):
        g = pl.ds((s * GPS + k) * GR, GR)
        lcps = [pltpu.async_copy(p0_hbm.at[g], ta, semt),
                pltpu.async_copy(p1_hbm.at[g], tb, semt),
                pltpu.async_copy(ro_hbm.at[g], tc_, semt)]
        for cp in lcps:
            cp.wait()

        @pl.loop(0, GR)
        def _addrow(i):
            for q in range(H // LANES):
                sl = pl.ds(q * LANES, LANES)
                rows0[i, sl] = rows0[i, sl] + rows0[GR + i, sl] \
                    + rows0[2 * GR + i, sl]

        pltpu.sync_copy(ta, out1_hbm.at[g])

    plsc.subcore_barrier()

    b_base = wid * WPW
    bufs = ((sdew0, srcw0, dstw0, rows0, semi0, semg0, sems0),
            (sdew1, srcw1, dstw1, rows1, semi1, semg1, sems1))

    def _prep(b, t):
        sdew, srcw, dstw, rows, semi, semg, _ = bufs[b]
        pltpu.make_async_copy(sde_hbm.at[t], sdew, semi).wait()
        for j in range(WIN_ROWS):
            for q in range(CW // LANES):
                sl = pl.ds(q * LANES, LANES)
                srcw[j, sl] = sdew[0, j, sl]
                dstw[j, sl] = sdew[1, j, sl]

        @pl.when(t + 2 < b_base + WPW)
        def _():
            pltpu.async_copy(sde_hbm.at[t + 2], sdew, semi)
        for j in range(WIN_ROWS):
            pltpu.async_copy(out1_hbm.at[srcw.at[j]],
                             rows.at[pl.ds(j * CW, CW)], semg)

    def _consume(b):
        _, srcw, dstw, rows, _, semg, sems = bufs[b]
        for j in range(WIN_ROWS):
            pltpu.make_async_copy(out1_hbm.at[srcw.at[j]],
                                  rows.at[pl.ds(j * CW, CW)], semg).wait()
            pltpu.async_copy(rows.at[pl.ds(j * CW, CW)],
                             acc_sp.at[dstw.at[j]], sems, add=True)

    def _wait_scat(b):
        _, _, dstw, rows, _, _, sems = bufs[b]
        for j in range(WIN_ROWS):
            pltpu.make_async_copy(rows.at[pl.ds(j * CW, CW)],
                                  acc_sp.at[dstw.at[j]], sems).wait()

    pltpu.async_copy(sde_hbm.at[b_base], sdew0, semi0)
    pltpu.async_copy(sde_hbm.at[b_base + 1], sdew1, semi1)
    _prep(0, b_base)

    @pl.loop(0, WPW // 2)
    def _edge_iter(i):
        t = b_base + 2 * i
        _prep(1, t + 1)
        _consume(0)
        _wait_scat(0)
        _prep(0, t + 2)
        _consume(1)
        _wait_scat(1)

    _consume(0)
    _wait_scat(0)

    plsc.subcore_barrier()

    slots = [rows0.at[pl.ds(0, GR)], rows0.at[pl.ds(GR, GR)],
             rows0.at[pl.ds(2 * GR, GR)], rows1.at[pl.ds(0, GR)],
             rows1.at[pl.ds(GR, GR)]]
    for k in range(GPS):
        pltpu.sync_copy(acc_sp.at[pl.ds(s * ACC_STRIPE + k * GR, GR)],
                        slots[k])
    for k in range(GPS):
        rs = pl.ds(s * ACC_STRIPE + k * GR, GR)

        @pl.when(c == 0)
        def _():
            pltpu.async_copy(slots[k], q0_hbm.at[rs], semz)

        @pl.when(c == 1)
        def _():
            pltpu.async_copy(slots[k], q1_hbm.at[rs], semz)
    for k in range(GPS):
        rs = pl.ds(s * ACC_STRIPE + k * GR, GR)
        pltpu.make_async_copy(slots[k], q0_hbm.at[rs], semz).wait()


def _sc_2(p0, p1, ro, sde):
    kern = pl.kernel(
        _sc2_body,
        out_type=[_f32(N, H), _f32(N, H), _f32(N, H)],
        mesh=_mesh,
        scratch_types=[
            pltpu.VMEM_SHARED((N, H), jnp.float32),
            pltpu.VMEM((3, WIN_ROWS, CW), jnp.int32),
            pltpu.VMEM((3, WIN_ROWS, CW), jnp.int32),
            pltpu.VMEM((WIN_ROWS, CW), jnp.int32),
            pltpu.VMEM((WIN_ROWS, CW), jnp.int32),
            pltpu.VMEM((WIN_ROWS, CW), jnp.int32),
            pltpu.VMEM((WIN_ROWS, CW), jnp.int32),
            pltpu.VMEM((WIN_E, H), jnp.float32),
            pltpu.VMEM((WIN_E, H), jnp.float32),
            pltpu.SemaphoreType.DMA,
            pltpu.SemaphoreType.DMA,
            pltpu.SemaphoreType.DMA,
            pltpu.SemaphoreType.DMA,
            pltpu.SemaphoreType.DMA,
            pltpu.SemaphoreType.DMA,
            pltpu.SemaphoreType.DMA,
            pltpu.SemaphoreType.DMA,
        ],
        compiler_params=_sc_params,
    )
    return kern(p0, p1, ro, sde)


def _tcb_body(x_ref, o1_ref, q0_ref, q1_ref, wrel_ref, brel_ref, wr2_ref,
              wlx_ref, wlo_ref, bl_ref, ws_ref, bs_ref, out_ref):
    agg = q0_ref[...] + q1_ref[...]
    out1 = o1_ref[...]
    out2 = (jnp.dot(agg, wrel_ref[...], preferred_element_type=jnp.float32)
            + jnp.dot(out1, wr2_ref[...], preferred_element_type=jnp.float32)
            + brel_ref[...])
    h = jnp.maximum(
        jnp.dot(x_ref[...], wlx_ref[...], preferred_element_type=jnp.float32)
        + jnp.dot(out2, wlo_ref[...], preferred_element_type=jnp.float32)
        + bl_ref[...], 0.0)
    lg = jnp.dot(h, ws_ref[...], preferred_element_type=jnp.float32) + bs_ref[...]
    m = jnp.max(lg, axis=1, keepdims=True)
    sh = lg - m
    out_ref[...] = sh - jnp.log(jnp.sum(jnp.exp(sh), axis=1, keepdims=True))


def _tc_b(x, out1, q0, q1, wrel, brel, wr2, wlx, wlo, bl, ws, bs):
    bn = 1000
    grid = (N // bn,)

    def full(shape):
        return pl.BlockSpec(shape, lambda i: tuple(0 for _ in shape))

    return pl.pallas_call(
        _tcb_body,
        grid=grid,
        in_specs=[
            pl.BlockSpec((bn, F_IN), lambda i: (i, 0)),
            pl.BlockSpec((bn, H), lambda i: (i, 0)),
            pl.BlockSpec((bn, H), lambda i: (i, 0)),
            pl.BlockSpec((bn, H), lambda i: (i, 0)),
            full((H, H)),
            full((1, H)),
            full((H, H)),
            full((F_IN, H)),
            full((H, H)),
            full((1, H)),
            full((H, NC)),
            full((1, NC)),
        ],
        out_specs=pl.BlockSpec((bn, NC), lambda i: (i, 0)),
        out_shape=_f32(N, NC),
    )(x, out1, q0, q1, wrel, brel, wr2, wlx, wlo, bl, ws, bs)


def kernel(x, edge_index, edge_norm, edge_type, seq_lengths, umask,
           nodal_attn, avec, bases, comp, root, bias1, W_rel, b_rel,
           W_root2, Wl, bl, Ws, bs):
    w2 = jnp.einsum("rb,bio->iro", comp, bases).reshape(F_IN, R * H)
    src3 = edge_index[0].astype(jnp.int32).reshape(NWINDOWS, WIN_ROWS, CW)
    dst3 = edge_index[1].astype(jnp.int32).reshape(NWINDOWS, WIN_ROWS, CW)
    et3 = edge_type.astype(jnp.int32).reshape(NWINDOWS, WIN_ROWS, CW)
    sde = jnp.stack([src3, dst3, et3], axis=1)

    xr, root_out = _tc_a(x, w2, root, bias1.reshape(1, H))
    xr2 = xr.reshape(N * R, H)

    p0, p1 = _sc_1(xr2, sde)

    out1, q0, q1 = _sc_2(p0, p1, root_out, sde)

    return _tc_b(x, out1, q0, q1, W_rel,
                 b_rel.reshape(1, H), W_root2, Wl[:F_IN], Wl[F_IN:],
                 bl.reshape(1, H), Ws, bs.reshape(1, NC))
